# baseline (speedup 1.0000x reference)
import functools

import jax
import jax.numpy as jnp
import numpy as np
from jax import lax
from jax.experimental import pallas as pl
from jax.experimental.pallas import tpu as pltpu
from jax.experimental.pallas import tpu_sc as plsc

_N = 50000
_T = 16
_H = 64
_OUT = 64

_BN = 512
_GA = 98
_NPAD = _BN * _GA
_SW = _NPAD // 16

_E = 800000
_EPT = 50176
_EPAD = 16 * _EPT
_SUP = 3584
_NSUP = _EPT // _SUP
_SUBS = _SUP // 128
_G16 = _SUP // 16

_SUP2 = 1792
_NSUP2 = _EPT // _SUP2
_SUBS2 = _SUP2 // 128
_G162 = _SUP2 // 16

_NC = 2
_NS = 16

_SC_PARAMS = pltpu.CompilerParams(needs_layout_passes=False,
                                  use_tc_tiling_on_sc=False)


def _sc1_body(src_hbm, dst_hbm, w_hbm,
              dis_hbm, s_hbm,
              dis_v, src_v, dst_v, w_v, val_v, dstb, srcb, stripe_v,
              deg_sh, s_sh):
    c = lax.axis_index("c")
    sid = lax.axis_index("s")
    ebase = sid * _EPT
    z16 = jnp.zeros((16,), jnp.float32)

    def _zstripe(i, carry):
        stripe_v[pl.ds(i * 16, 16)] = z16
        return carry
    lax.fori_loop(0, _SW // 16, _zstripe, None)
    pltpu.sync_copy(stripe_v, deg_sh.at[pl.ds(sid * _SW, _SW)])
    pltpu.sync_copy(stripe_v, s_sh.at[pl.ds(sid * _SW, _SW)])
    plsc.subcore_barrier()

    def _deg_super(ks, carry):
        base = ebase + ks * _SUP
        pltpu.sync_copy(dst_hbm.at[pl.ds(base, _SUP)], dst_v)
        pltpu.sync_copy(w_hbm.at[pl.ds(base, _SUP)], w_v)

        def _sub(j, carry2):
            def _cp(g, carry3):
                dstb[pl.ds(g * 16, 16)] = dst_v[pl.ds(j * 128 + g * 16, 16)]
                return carry3
            lax.fori_loop(0, 8, _cp, None)
            pltpu.sync_copy(w_v.at[pl.ds(j * 128, 128)],
                            deg_sh.at[dstb], add=True)
            return carry2
        lax.fori_loop(0, _SUBS, _sub, None)
        return carry
    lax.fori_loop(0, _NSUP, _deg_super, None)
    plsc.subcore_barrier()

    pltpu.sync_copy(deg_sh.at[pl.ds(sid * _SW, _SW)], stripe_v)

    def _newton(i, carry):
        v = stripe_v[pl.ds(i * 16, 16)] + 1.0
        iv = lax.bitcast_convert_type(v, jnp.int32)
        yi = jnp.int32(0x5F3759DF) - lax.shift_right_logical(iv, 1)
        y = lax.bitcast_convert_type(yi, jnp.float32)
        y = y * (1.5 - 0.5 * v * y * y)
        y = y * (1.5 - 0.5 * v * y * y)
        y = y * (1.5 - 0.5 * v * y * y)
        y = y * (1.5 - 0.5 * v * y * y)
        stripe_v[pl.ds(i * 16, 16)] = y
        return carry
    lax.fori_loop(0, _SW // 16, _newton, None)
    pltpu.sync_copy(stripe_v, deg_sh.at[pl.ds(sid * _SW, _SW)])

    @pl.when(c == 0)
    def _():
        pltpu.sync_copy(stripe_v, dis_hbm.at[pl.ds(sid * _SW, _SW)])
    plsc.subcore_barrier()

    pltpu.sync_copy(deg_sh, dis_v)

    def _s_super(ks, carry):
        base = ebase + ks * _SUP
        pltpu.sync_copy(src_hbm.at[pl.ds(base, _SUP)], src_v)
        pltpu.sync_copy(dst_hbm.at[pl.ds(base, _SUP)], dst_v)
        pltpu.sync_copy(w_hbm.at[pl.ds(base, _SUP)], w_v)

        def _val(g, carry2):
            d16 = dst_v[pl.ds(g * 16, 16)]
            w16 = w_v[pl.ds(g * 16, 16)]
            disd = plsc.load_gather(dis_v, [d16])
            val_v[pl.ds(g * 16, 16)] = w16 * disd
            return carry2
        lax.fori_loop(0, _G16, _val, None)

        def _sub(j, carry2):
            def _cp(g, carry3):
                srcb[pl.ds(g * 16, 16)] = src_v[pl.ds(j * 128 + g * 16, 16)]
                return carry3
            lax.fori_loop(0, 8, _cp, None)
            pltpu.sync_copy(val_v.at[pl.ds(j * 128, 128)],
                            s_sh.at[srcb], add=True)
            return carry2
        lax.fori_loop(0, _SUBS, _sub, None)
        return carry
    lax.fori_loop(0, _NSUP, _s_super, None)
    plsc.subcore_barrier()

    @pl.when(c == 0)
    def _():
        pltpu.sync_copy(s_sh.at[pl.ds(sid * _SW, _SW)],
                        s_hbm.at[pl.ds(sid * _SW, _SW)])


_sc1_stage = functools.partial(
    pl.kernel,
    out_type=[
        jax.ShapeDtypeStruct((_NPAD,), jnp.float32),
        jax.ShapeDtypeStruct((_NPAD,), jnp.float32),
    ],
    mesh=plsc.VectorSubcoreMesh(core_axis_name="c", subcore_axis_name="s",
                                num_cores=_NC, num_subcores=_NS),
    compiler_params=_SC_PARAMS,
    scratch_types=[
        pltpu.VMEM((_NPAD,), jnp.float32),
        pltpu.VMEM((_SUP,), jnp.int32),
        pltpu.VMEM((_SUP,), jnp.int32),
        pltpu.VMEM((_SUP,), jnp.float32),
        pltpu.VMEM((_SUP,), jnp.float32),
        pltpu.VMEM((128,), jnp.int32),
        pltpu.VMEM((128,), jnp.int32),
        pltpu.VMEM((_SW,), jnp.float32),
        pltpu.VMEM_SHARED((_NPAD,), jnp.float32),
        pltpu.VMEM_SHARED((_NPAD,), jnp.float32),
    ],
)(_sc1_body)


def _lstm_body(x_ref, wih_ref, whhT_ref, b_ref, w1T_ref, dis_ref, out_ref):
    x = x_ref[...]
    wih = wih_ref[...]
    b = b_ref[...]
    whhT = whhT_ref[...].astype(jnp.bfloat16)
    h = jnp.zeros((_BN, _H), jnp.float32)
    c = jnp.zeros((_BN, _H), jnp.float32)
    for t in range(_T):
        xt = x[:, t:t + 1]
        g = xt * wih + b
        g = g + jnp.dot(h.astype(jnp.bfloat16), whhT,
                        preferred_element_type=jnp.float32)
        i = jax.nn.sigmoid(g[:, 0:_H])
        f = jax.nn.sigmoid(g[:, _H:2 * _H])
        gg = jnp.tanh(g[:, 2 * _H:3 * _H])
        o = jax.nn.sigmoid(g[:, 3 * _H:4 * _H])
        c = f * c + i * gg
        h = o * jnp.tanh(c)
    xw = jnp.dot(jax.nn.relu(h), w1T_ref[...],
                 preferred_element_type=jnp.float32)
    xw = xw * dis_ref[...]
    out_ref[0, :, :] = xw[:, 0:32]
    out_ref[1, :, :] = xw[:, 32:64]


def _lstm_stage(xpad, wih, whhT, b, w1T, dis):
    return pl.pallas_call(
        _lstm_body,
        grid=(_GA,),
        in_specs=[
            pl.BlockSpec((_BN, _T), lambda i: (i, 0)),
            pl.BlockSpec((1, 4 * _H), lambda i: (0, 0)),
            pl.BlockSpec((_H, 4 * _H), lambda i: (0, 0)),
            pl.BlockSpec((1, 4 * _H), lambda i: (0, 0)),
            pl.BlockSpec((_H, _H), lambda i: (0, 0)),
            pl.BlockSpec((_BN, 1), lambda i: (i, 0)),
        ],
        out_specs=pl.BlockSpec((2, _BN, 32), lambda i: (0, i, 0)),
        out_shape=jax.ShapeDtypeStruct((2, _NPAD, 32), jnp.float32),
    )(xpad, wih, whhT, b, w1T, dis)


def _sc2_body(src_hbm, dst_hbm, w_hbm, xw1_hbm,
              y1_hbm,
              src_v, dst_v, w_v, idx_v, rows_a, rows_b, dstb_a, dstb_b,
              y1_sh, gsem, ssem):
    c = lax.axis_index("c")
    sid = lax.axis_index("s")
    ebase = sid * _EPT
    cN = c * _NPAD
    z16 = jnp.zeros((16,), jnp.float32)
    bufs = (rows_a, rows_b)
    dbufs = (dstb_a, dstb_b)

    def _zrows(e, carry):
        rows_a[e, pl.ds(0, 16)] = z16
        rows_a[e, pl.ds(16, 16)] = z16
        return carry
    lax.fori_loop(0, 128, _zrows, None)

    def _zy(k, carry):
        pltpu.sync_copy(rows_a, y1_sh.at[pl.ds(sid * _SW + k * 128, 128)])
        return carry
    lax.fori_loop(0, _SW // 128, _zy, None)
    pltpu.sync_copy(rows_a.at[pl.ds(0, 64)],
                    y1_sh.at[pl.ds(sid * _SW + (_SW // 128) * 128, 64)])
    plsc.subcore_barrier()

    def _main_super(ks, carry):
        base = ebase + ks * _SUP2
        pltpu.sync_copy(src_hbm.at[pl.ds(base, _SUP2)], src_v)
        pltpu.sync_copy(dst_hbm.at[pl.ds(base, _SUP2)], dst_v)
        pltpu.sync_copy(w_hbm.at[pl.ds(base, _SUP2)], w_v)

        def _page(g, carry2):
            idx_v[pl.ds(g * 16, 16)] = src_v[pl.ds(g * 16, 16)] + cN
            return carry2
        lax.fori_loop(0, _G162, _page, None)

        sdesc = [None, None]
        gdesc = [None, None]
        gdesc[0] = pltpu.async_copy(
            xw1_hbm.at[idx_v.at[pl.ds(0, 128)]], rows_a, gsem)
        for j in range(_SUBS2):
            p = j & 1
            q = 1 - p
            if j + 1 < _SUBS2:
                if sdesc[q] is not None:
                    sdesc[q].wait()
                    sdesc[q] = None
                gdesc[q] = pltpu.async_copy(
                    xw1_hbm.at[idx_v.at[pl.ds((j + 1) * 128, 128)]],
                    bufs[q], gsem)
            gdesc[p].wait()

            def _scale(g, carry3, _j=j, _p=p):
                w16 = w_v[pl.ds(_j * 128 + g * 16, 16)]
                for k in range(16):
                    wk = w16[k]
                    r = bufs[_p]
                    e = g * 16 + k
                    r[e, pl.ds(0, 16)] = r[e, pl.ds(0, 16)] * wk
                    r[e, pl.ds(16, 16)] = r[e, pl.ds(16, 16)] * wk
                return carry3
            lax.fori_loop(0, 8, _scale, None)

            def _cp(g, carry3, _j=j, _p=p):
                dbufs[_p][pl.ds(g * 16, 16)] = \
                    dst_v[pl.ds(_j * 128 + g * 16, 16)]
                return carry3
            lax.fori_loop(0, 8, _cp, None)
            sdesc[p] = pltpu.async_copy(bufs[p], y1_sh.at[dbufs[p]],
                                        ssem, add=True)
        for d in sdesc:
            if d is not None:
                d.wait()
        return carry
    lax.fori_loop(0, _NSUP2, _main_super, None)
    plsc.subcore_barrier()

    def _wb(k, carry):
        pltpu.sync_copy(y1_sh.at[pl.ds(sid * _SW + k * 1024, 1024)],
                        y1_hbm.at[pl.ds(cN + sid * _SW + k * 1024, 1024)])
        return carry
    lax.fori_loop(0, 3, _wb, None)
    pltpu.sync_copy(y1_sh.at[pl.ds(sid * _SW + 3072, 64)],
                    y1_hbm.at[pl.ds(cN + sid * _SW + 3072, 64)])


_sc2_stage = functools.partial(
    pl.kernel,
    out_type=[
        jax.ShapeDtypeStruct((2 * _NPAD, 32), jnp.float32),
    ],
    mesh=plsc.VectorSubcoreMesh(core_axis_name="c", subcore_axis_name="s",
                                num_cores=_NC, num_subcores=_NS),
    compiler_params=_SC_PARAMS,
    scratch_types=[
        pltpu.VMEM((_SUP2,), jnp.int32),
        pltpu.VMEM((_SUP2,), jnp.int32),
        pltpu.VMEM((_SUP2,), jnp.float32),
        pltpu.VMEM((_SUP2,), jnp.int32),
        pltpu.VMEM((128, 32), jnp.float32),
        pltpu.VMEM((128, 32), jnp.float32),
        pltpu.VMEM((128,), jnp.int32),
        pltpu.VMEM((128,), jnp.int32),
        pltpu.VMEM_SHARED((_NPAD, 32), jnp.float32),
        pltpu.SemaphoreType.DMA,
        pltpu.SemaphoreType.DMA,
    ],
)(_sc2_body)


def _final_body(y1_ref, xw1_ref, dis_ref, s_ref, b1_ref, w2T_ref, b2_ref,
                out_ref, acc_ref):
    i = pl.program_id(0)

    @pl.when(i == 0)
    def _():
        acc_ref[...] = jnp.zeros_like(acc_ref)

    dis = dis_ref[...]
    sv = s_ref[...]
    rowid = i * _BN + lax.broadcasted_iota(jnp.int32, (_BN, 1), 0)
    mask = rowid < _N
    a = jnp.where(mask, dis * (sv + dis), 0.0)
    y1 = jnp.concatenate([y1_ref[0], y1_ref[1]], axis=1)
    xw1 = jnp.concatenate([xw1_ref[0], xw1_ref[1]], axis=1)
    z = jax.nn.relu(dis * (y1 + xw1) + b1_ref[...])
    z = jnp.where(mask, z, 0.0)
    acc_ref[...] += jnp.sum(z * a, axis=0, keepdims=True)

    @pl.when(i == _GA - 1)
    def _():
        out_ref[...] = (jnp.dot(acc_ref[...], w2T_ref[...],
                                preferred_element_type=jnp.float32)
                        * np.float32(1.0 / _N) + b2_ref[...])


def _final_stage(y1p, xw1p, dis, s, b1, w2T, b2):
    return pl.pallas_call(
        _final_body,
        grid=(_GA,),
        in_specs=[
            pl.BlockSpec((2, _BN, 32), lambda i: (0, i, 0)),
            pl.BlockSpec((2, _BN, 32), lambda i: (0, i, 0)),
            pl.BlockSpec((_BN, 1), lambda i: (i, 0)),
            pl.BlockSpec((_BN, 1), lambda i: (i, 0)),
            pl.BlockSpec((1, _H), lambda i: (0, 0)),
            pl.BlockSpec((_H, _OUT), lambda i: (0, 0)),
            pl.BlockSpec((1, _OUT), lambda i: (0, 0)),
        ],
        out_specs=pl.BlockSpec((1, _OUT), lambda i: (0, 0)),
        out_shape=jax.ShapeDtypeStruct((1, _OUT), jnp.float32),
        scratch_shapes=[pltpu.VMEM((1, _OUT), jnp.float32)],
    )(y1p, xw1p, dis, s, b1, w2T, b2)


def kernel(node_features, edge_index, edge_attributes,
           W_ih, W_hh, b_ih, b_hh, W1, b1, W2, b2):
    f32 = jnp.float32
    xpad = jnp.zeros((_NPAD, _T), f32).at[:_N].set(node_features)
    wih = W_ih[:, 0][None, :]
    whhT = W_hh.T
    bsum = (b_ih + b_hh)[None, :]
    w1T = W1.T

    epad = _EPAD - _E
    zi = jnp.zeros((epad,), jnp.int32)
    srcp = jnp.concatenate([edge_index[0], zi])
    dstp = jnp.concatenate([edge_index[1], zi])
    wp = jnp.concatenate([edge_attributes, jnp.zeros((epad,), f32)])

    dis, s = _sc1_stage(srcp, dstp, wp)

    xw1p = _lstm_stage(xpad, wih, whhT, bsum, w1T, dis[:, None])
    xw1flat = xw1p.reshape(2 * _NPAD, 32)

    (y1flat,) = _sc2_stage(srcp, dstp, wp, xw1flat)
    y1p = y1flat.reshape(2, _NPAD, 32)

    return _final_stage(y1p, xw1p, dis[:, None], s[:, None],
                        b1[None, :], W2.T, b2[None, :])


# speedup vs baseline: 1.0088x; 1.0088x over previous
import functools

import jax
import jax.numpy as jnp
import numpy as np
from jax import lax
from jax.experimental import pallas as pl
from jax.experimental.pallas import tpu as pltpu
from jax.experimental.pallas import tpu_sc as plsc

_N = 50000
_T = 16
_H = 64
_OUT = 64

_BN = 512
_GA = 98
_NPAD = _BN * _GA
_SW = _NPAD // 16

_E = 800000
_EPT = 50176
_EPAD = 16 * _EPT
_SUP = 3584
_NSUP = _EPT // _SUP
_SUBS = _SUP // 128
_G16 = _SUP // 16

_SUP2 = 1792
_NSUP2 = _EPT // _SUP2
_SUBS2 = _SUP2 // 128
_G162 = _SUP2 // 16

_NC = 2
_NS = 16

_SC_PARAMS = pltpu.CompilerParams(needs_layout_passes=False,
                                  use_tc_tiling_on_sc=False)


def _sc1_body(src_hbm, dst_hbm, w_hbm,
              dis_hbm, s_hbm,
              dis_v, src_v, dst_v, w_v, val_v, dstb, srcb, stripe_v,
              deg_sh, s_sh):
    c = lax.axis_index("c")
    sid = lax.axis_index("s")
    ebase = sid * _EPT
    z16 = jnp.zeros((16,), jnp.float32)

    def _zstripe(i, carry):
        stripe_v[pl.ds(i * 16, 16)] = z16
        return carry
    lax.fori_loop(0, _SW // 16, _zstripe, None)
    pltpu.sync_copy(stripe_v, deg_sh.at[pl.ds(sid * _SW, _SW)])
    pltpu.sync_copy(stripe_v, s_sh.at[pl.ds(sid * _SW, _SW)])
    plsc.subcore_barrier()

    def _deg_super(ks, carry):
        base = ebase + ks * _SUP
        pltpu.sync_copy(dst_hbm.at[pl.ds(base, _SUP)], dst_v)
        pltpu.sync_copy(w_hbm.at[pl.ds(base, _SUP)], w_v)

        def _sub(j, carry2):
            def _cp(g, carry3):
                dstb[pl.ds(g * 16, 16)] = dst_v[pl.ds(j * 128 + g * 16, 16)]
                return carry3
            lax.fori_loop(0, 8, _cp, None)
            pltpu.sync_copy(w_v.at[pl.ds(j * 128, 128)],
                            deg_sh.at[dstb], add=True)
            return carry2
        lax.fori_loop(0, _SUBS, _sub, None)
        return carry
    lax.fori_loop(0, _NSUP, _deg_super, None)
    plsc.subcore_barrier()

    pltpu.sync_copy(deg_sh.at[pl.ds(sid * _SW, _SW)], stripe_v)

    def _newton(i, carry):
        v = stripe_v[pl.ds(i * 16, 16)] + 1.0
        iv = lax.bitcast_convert_type(v, jnp.int32)
        yi = jnp.int32(0x5F3759DF) - lax.shift_right_logical(iv, 1)
        y = lax.bitcast_convert_type(yi, jnp.float32)
        y = y * (1.5 - 0.5 * v * y * y)
        y = y * (1.5 - 0.5 * v * y * y)
        y = y * (1.5 - 0.5 * v * y * y)
        y = y * (1.5 - 0.5 * v * y * y)
        stripe_v[pl.ds(i * 16, 16)] = y
        return carry
    lax.fori_loop(0, _SW // 16, _newton, None)
    pltpu.sync_copy(stripe_v, deg_sh.at[pl.ds(sid * _SW, _SW)])

    @pl.when(c == 0)
    def _():
        pltpu.sync_copy(stripe_v, dis_hbm.at[pl.ds(sid * _SW, _SW)])
    plsc.subcore_barrier()

    pltpu.sync_copy(deg_sh, dis_v)

    def _s_super(ks, carry):
        base = ebase + ks * _SUP
        pltpu.sync_copy(src_hbm.at[pl.ds(base, _SUP)], src_v)
        pltpu.sync_copy(dst_hbm.at[pl.ds(base, _SUP)], dst_v)
        pltpu.sync_copy(w_hbm.at[pl.ds(base, _SUP)], w_v)

        def _val(g, carry2):
            d16 = dst_v[pl.ds(g * 16, 16)]
            w16 = w_v[pl.ds(g * 16, 16)]
            disd = plsc.load_gather(dis_v, [d16])
            val_v[pl.ds(g * 16, 16)] = w16 * disd
            return carry2
        lax.fori_loop(0, _G16, _val, None)

        def _sub(j, carry2):
            def _cp(g, carry3):
                srcb[pl.ds(g * 16, 16)] = src_v[pl.ds(j * 128 + g * 16, 16)]
                return carry3
            lax.fori_loop(0, 8, _cp, None)
            pltpu.sync_copy(val_v.at[pl.ds(j * 128, 128)],
                            s_sh.at[srcb], add=True)
            return carry2
        lax.fori_loop(0, _SUBS, _sub, None)
        return carry
    lax.fori_loop(0, _NSUP, _s_super, None)
    plsc.subcore_barrier()

    @pl.when(c == 0)
    def _():
        pltpu.sync_copy(s_sh.at[pl.ds(sid * _SW, _SW)],
                        s_hbm.at[pl.ds(sid * _SW, _SW)])


_sc1_stage = functools.partial(
    pl.kernel,
    out_type=[
        jax.ShapeDtypeStruct((_NPAD,), jnp.float32),
        jax.ShapeDtypeStruct((_NPAD,), jnp.float32),
    ],
    mesh=plsc.VectorSubcoreMesh(core_axis_name="c", subcore_axis_name="s",
                                num_cores=_NC, num_subcores=_NS),
    compiler_params=_SC_PARAMS,
    scratch_types=[
        pltpu.VMEM((_NPAD,), jnp.float32),
        pltpu.VMEM((_SUP,), jnp.int32),
        pltpu.VMEM((_SUP,), jnp.int32),
        pltpu.VMEM((_SUP,), jnp.float32),
        pltpu.VMEM((_SUP,), jnp.float32),
        pltpu.VMEM((128,), jnp.int32),
        pltpu.VMEM((128,), jnp.int32),
        pltpu.VMEM((_SW,), jnp.float32),
        pltpu.VMEM_SHARED((_NPAD,), jnp.float32),
        pltpu.VMEM_SHARED((_NPAD,), jnp.float32),
    ],
)(_sc1_body)


def _lstm_body(x_ref, wih_ref, whhT_ref, b_ref, w1T_ref, dis_ref, out_ref):
    x = x_ref[...]
    wih = wih_ref[...]
    b = b_ref[...]
    whhT = whhT_ref[...].astype(jnp.bfloat16)
    h = jnp.zeros((_BN, _H), jnp.float32)
    c = jnp.zeros((_BN, _H), jnp.float32)
    for t in range(_T):
        xt = x[:, t:t + 1]
        g = xt * wih + b
        g = g + jnp.dot(h.astype(jnp.bfloat16), whhT,
                        preferred_element_type=jnp.float32)
        i = jax.nn.sigmoid(g[:, 0:_H])
        f = jax.nn.sigmoid(g[:, _H:2 * _H])
        gg = jnp.tanh(g[:, 2 * _H:3 * _H])
        o = jax.nn.sigmoid(g[:, 3 * _H:4 * _H])
        c = f * c + i * gg
        h = o * jnp.tanh(c)
    xw = jnp.dot(jax.nn.relu(h), w1T_ref[...],
                 preferred_element_type=jnp.float32)
    xw = xw * dis_ref[...]
    out_ref[0, :, :] = xw[:, 0:32]
    out_ref[1, :, :] = xw[:, 32:64]


def _lstm_stage(xpad, wih, whhT, b, w1T, dis):
    return pl.pallas_call(
        _lstm_body,
        grid=(_GA,),
        in_specs=[
            pl.BlockSpec((_BN, _T), lambda i: (i, 0)),
            pl.BlockSpec((1, 4 * _H), lambda i: (0, 0)),
            pl.BlockSpec((_H, 4 * _H), lambda i: (0, 0)),
            pl.BlockSpec((1, 4 * _H), lambda i: (0, 0)),
            pl.BlockSpec((_H, _H), lambda i: (0, 0)),
            pl.BlockSpec((_BN, 1), lambda i: (i, 0)),
        ],
        out_specs=pl.BlockSpec((2, _BN, 32), lambda i: (0, i, 0)),
        out_shape=jax.ShapeDtypeStruct((2, _NPAD, 32), jnp.float32),
    )(xpad, wih, whhT, b, w1T, dis)


def _sc2_body(src_hbm, dst_hbm, w_hbm, xw1_hbm,
              y1_hbm,
              src_v, dst_v, w_v, rows_a, rows_b, dstb_a, dstb_b,
              y1_sh, gsem, ssem):
    c = lax.axis_index("c")
    sid = lax.axis_index("s")
    ebase = sid * _EPT
    z16 = jnp.zeros((16,), jnp.float32)
    bufs = (rows_a, rows_b)
    dbufs = (dstb_a, dstb_b)
    xw1_page = xw1_hbm.at[c]
    y1_page = y1_hbm.at[c]

    def _zrows(e, carry):
        rows_a[e, pl.ds(0, 16)] = z16
        rows_a[e, pl.ds(16, 16)] = z16
        return carry
    lax.fori_loop(0, 128, _zrows, None)

    def _zy(k, carry):
        pltpu.sync_copy(rows_a, y1_sh.at[pl.ds(sid * _SW + k * 128, 128)])
        return carry
    lax.fori_loop(0, _SW // 128, _zy, None)
    pltpu.sync_copy(rows_a.at[pl.ds(0, 64)],
                    y1_sh.at[pl.ds(sid * _SW + (_SW // 128) * 128, 64)])
    plsc.subcore_barrier()

    def _main_super(ks, carry):
        base = ebase + ks * _SUP2
        pltpu.sync_copy(src_hbm.at[pl.ds(base, _SUP2)], src_v)
        pltpu.sync_copy(dst_hbm.at[pl.ds(base, _SUP2)], dst_v)
        pltpu.sync_copy(w_hbm.at[pl.ds(base, _SUP2)], w_v)

        sdesc = [None, None]
        gdesc = [None, None]
        gdesc[0] = pltpu.async_copy(
            xw1_page.at[src_v.at[pl.ds(0, 128)]], rows_a, gsem)
        for j in range(_SUBS2):
            p = j & 1
            q = 1 - p
            if j + 1 < _SUBS2:
                if sdesc[q] is not None:
                    sdesc[q].wait()
                    sdesc[q] = None
                gdesc[q] = pltpu.async_copy(
                    xw1_page.at[src_v.at[pl.ds((j + 1) * 128, 128)]],
                    bufs[q], gsem)
            gdesc[p].wait()

            def _scale(g, carry3, _j=j, _p=p):
                w16 = w_v[pl.ds(_j * 128 + g * 16, 16)]
                for k in range(16):
                    wk = w16[k]
                    r = bufs[_p]
                    e = g * 16 + k
                    r[e, pl.ds(0, 16)] = r[e, pl.ds(0, 16)] * wk
                    r[e, pl.ds(16, 16)] = r[e, pl.ds(16, 16)] * wk
                return carry3
            lax.fori_loop(0, 8, _scale, None)

            def _cp(g, carry3, _j=j, _p=p):
                dbufs[_p][pl.ds(g * 16, 16)] = \
                    dst_v[pl.ds(_j * 128 + g * 16, 16)]
                return carry3
            lax.fori_loop(0, 8, _cp, None)
            sdesc[p] = pltpu.async_copy(bufs[p], y1_sh.at[dbufs[p]],
                                        ssem, add=True)
        for d in sdesc:
            if d is not None:
                d.wait()
        return carry
    lax.fori_loop(0, _NSUP2, _main_super, None)
    plsc.subcore_barrier()

    def _wb(k, carry):
        pltpu.sync_copy(y1_sh.at[pl.ds(sid * _SW + k * 1024, 1024)],
                        y1_page.at[pl.ds(sid * _SW + k * 1024, 1024)])
        return carry
    lax.fori_loop(0, 3, _wb, None)
    pltpu.sync_copy(y1_sh.at[pl.ds(sid * _SW + 3072, 64)],
                    y1_page.at[pl.ds(sid * _SW + 3072, 64)])


_sc2_stage = functools.partial(
    pl.kernel,
    out_type=[
        jax.ShapeDtypeStruct((2, _NPAD, 32), jnp.float32),
    ],
    mesh=plsc.VectorSubcoreMesh(core_axis_name="c", subcore_axis_name="s",
                                num_cores=_NC, num_subcores=_NS),
    compiler_params=_SC_PARAMS,
    scratch_types=[
        pltpu.VMEM((_SUP2,), jnp.int32),
        pltpu.VMEM((_SUP2,), jnp.int32),
        pltpu.VMEM((_SUP2,), jnp.float32),
        pltpu.VMEM((128, 32), jnp.float32),
        pltpu.VMEM((128, 32), jnp.float32),
        pltpu.VMEM((128,), jnp.int32),
        pltpu.VMEM((128,), jnp.int32),
        pltpu.VMEM_SHARED((_NPAD, 32), jnp.float32),
        pltpu.SemaphoreType.DMA,
        pltpu.SemaphoreType.DMA,
    ],
)(_sc2_body)


def _final_body(y1_ref, xw1_ref, dis_ref, s_ref, b1_ref, w2T_ref, b2_ref,
                out_ref, acc_ref):
    i = pl.program_id(0)

    @pl.when(i == 0)
    def _():
        acc_ref[...] = jnp.zeros_like(acc_ref)

    dis = dis_ref[...]
    sv = s_ref[...]
    rowid = i * _BN + lax.broadcasted_iota(jnp.int32, (_BN, 1), 0)
    mask = rowid < _N
    a = jnp.where(mask, dis * (sv + dis), 0.0)
    y1 = jnp.concatenate([y1_ref[0], y1_ref[1]], axis=1)
    xw1 = jnp.concatenate([xw1_ref[0], xw1_ref[1]], axis=1)
    z = jax.nn.relu(dis * (y1 + xw1) + b1_ref[...])
    z = jnp.where(mask, z, 0.0)
    acc_ref[...] += jnp.sum(z * a, axis=0, keepdims=True)

    @pl.when(i == _GA - 1)
    def _():
        out_ref[...] = (jnp.dot(acc_ref[...], w2T_ref[...],
                                preferred_element_type=jnp.float32)
                        * np.float32(1.0 / _N) + b2_ref[...])


def _final_stage(y1p, xw1p, dis, s, b1, w2T, b2):
    return pl.pallas_call(
        _final_body,
        grid=(_GA,),
        in_specs=[
            pl.BlockSpec((2, _BN, 32), lambda i: (0, i, 0)),
            pl.BlockSpec((2, _BN, 32), lambda i: (0, i, 0)),
            pl.BlockSpec((_BN, 1), lambda i: (i, 0)),
            pl.BlockSpec((_BN, 1), lambda i: (i, 0)),
            pl.BlockSpec((1, _H), lambda i: (0, 0)),
            pl.BlockSpec((_H, _OUT), lambda i: (0, 0)),
            pl.BlockSpec((1, _OUT), lambda i: (0, 0)),
        ],
        out_specs=pl.BlockSpec((1, _OUT), lambda i: (0, 0)),
        out_shape=jax.ShapeDtypeStruct((1, _OUT), jnp.float32),
        scratch_shapes=[pltpu.VMEM((1, _OUT), jnp.float32)],
    )(y1p, xw1p, dis, s, b1, w2T, b2)


def kernel(node_features, edge_index, edge_attributes,
           W_ih, W_hh, b_ih, b_hh, W1, b1, W2, b2):
    f32 = jnp.float32
    xpad = jnp.zeros((_NPAD, _T), f32).at[:_N].set(node_features)
    wih = W_ih[:, 0][None, :]
    whhT = W_hh.T
    bsum = (b_ih + b_hh)[None, :]
    w1T = W1.T

    epad = _EPAD - _E
    zi = jnp.zeros((epad,), jnp.int32)
    srcp = jnp.concatenate([edge_index[0], zi])
    dstp = jnp.concatenate([edge_index[1], zi])
    wp = jnp.concatenate([edge_attributes, jnp.zeros((epad,), f32)])

    dis, s = _sc1_stage(srcp, dstp, wp)

    xw1p = _lstm_stage(xpad, wih, whhT, bsum, w1T, dis[:, None])

    (y1p,) = _sc2_stage(srcp, dstp, wp, xw1p)

    return _final_stage(y1p, xw1p, dis[:, None], s[:, None],
                        b1[None, :], W2.T, b2[None, :])


# speedup vs baseline: 1.1388x; 1.1288x over previous
import functools

import jax
import jax.numpy as jnp
import numpy as np
from jax import lax
from jax.experimental import pallas as pl
from jax.experimental.pallas import tpu as pltpu
from jax.experimental.pallas import tpu_sc as plsc

_N = 50000
_T = 16
_H = 64
_OUT = 64

_BN = 512
_GA = 98
_NPAD = _BN * _GA
_SW = _NPAD // 16

_E = 800000
_EPT = 50176
_EPAD = 16 * _EPT
_SUP = 3584
_NSUP = _EPT // _SUP
_SUBS = _SUP // 128
_G16 = _SUP // 16

_SUP2 = 1792
_NSUP2 = _EPT // _SUP2
_SUBS2 = _SUP2 // 128
_G162 = _SUP2 // 16

_NC = 2
_NS = 16

_SC_PARAMS = pltpu.CompilerParams(needs_layout_passes=False,
                                  use_tc_tiling_on_sc=False)


def _sc1_body(src_hbm, dst_hbm, w_hbm,
              dis_hbm, s_hbm, wsc_hbm,
              dis_v, src_v, dst_v, w_v, val_v, wsc_v, dstb, srcb, stripe_v,
              deg_sh, s_sh):
    c = lax.axis_index("c")
    sid = lax.axis_index("s")
    ebase = sid * _EPT
    z16 = jnp.zeros((16,), jnp.float32)

    def _zstripe(i, carry):
        stripe_v[pl.ds(i * 16, 16)] = z16
        return carry
    lax.fori_loop(0, _SW // 16, _zstripe, None)
    pltpu.sync_copy(stripe_v, deg_sh.at[pl.ds(sid * _SW, _SW)])
    pltpu.sync_copy(stripe_v, s_sh.at[pl.ds(sid * _SW, _SW)])
    plsc.subcore_barrier()

    def _deg_super(ks, carry):
        base = ebase + ks * _SUP
        pltpu.sync_copy(dst_hbm.at[pl.ds(base, _SUP)], dst_v)
        pltpu.sync_copy(w_hbm.at[pl.ds(base, _SUP)], w_v)

        def _sub(j, carry2):
            def _cp(g, carry3):
                dstb[pl.ds(g * 16, 16)] = dst_v[pl.ds(j * 128 + g * 16, 16)]
                return carry3
            lax.fori_loop(0, 8, _cp, None)
            pltpu.sync_copy(w_v.at[pl.ds(j * 128, 128)],
                            deg_sh.at[dstb], add=True)
            return carry2
        lax.fori_loop(0, _SUBS, _sub, None)
        return carry
    lax.fori_loop(0, _NSUP, _deg_super, None)
    plsc.subcore_barrier()

    pltpu.sync_copy(deg_sh.at[pl.ds(sid * _SW, _SW)], stripe_v)

    def _newton(i, carry):
        v = stripe_v[pl.ds(i * 16, 16)] + 1.0
        iv = lax.bitcast_convert_type(v, jnp.int32)
        yi = jnp.int32(0x5F3759DF) - lax.shift_right_logical(iv, 1)
        y = lax.bitcast_convert_type(yi, jnp.float32)
        y = y * (1.5 - 0.5 * v * y * y)
        y = y * (1.5 - 0.5 * v * y * y)
        y = y * (1.5 - 0.5 * v * y * y)
        y = y * (1.5 - 0.5 * v * y * y)
        stripe_v[pl.ds(i * 16, 16)] = y
        return carry
    lax.fori_loop(0, _SW // 16, _newton, None)
    pltpu.sync_copy(stripe_v, deg_sh.at[pl.ds(sid * _SW, _SW)])

    @pl.when(c == 0)
    def _():
        pltpu.sync_copy(stripe_v, dis_hbm.at[pl.ds(sid * _SW, _SW)])
    plsc.subcore_barrier()

    pltpu.sync_copy(deg_sh, dis_v)

    def _s_super(ks, carry):
        base = ebase + ks * _SUP
        pltpu.sync_copy(src_hbm.at[pl.ds(base, _SUP)], src_v)
        pltpu.sync_copy(dst_hbm.at[pl.ds(base, _SUP)], dst_v)
        pltpu.sync_copy(w_hbm.at[pl.ds(base, _SUP)], w_v)

        def _val(g, carry2):
            d16 = dst_v[pl.ds(g * 16, 16)]
            s16 = src_v[pl.ds(g * 16, 16)]
            w16 = w_v[pl.ds(g * 16, 16)]
            disd = plsc.load_gather(dis_v, [d16])
            diss = plsc.load_gather(dis_v, [s16])
            val_v[pl.ds(g * 16, 16)] = w16 * disd
            wsc_v[pl.ds(g * 16, 16)] = w16 * diss
            return carry2
        lax.fori_loop(0, _G16, _val, None)

        @pl.when(c == 0)
        def _():
            pltpu.sync_copy(wsc_v, wsc_hbm.at[pl.ds(base, _SUP)])

        def _sub(j, carry2):
            def _cp(g, carry3):
                srcb[pl.ds(g * 16, 16)] = src_v[pl.ds(j * 128 + g * 16, 16)]
                return carry3
            lax.fori_loop(0, 8, _cp, None)
            pltpu.sync_copy(val_v.at[pl.ds(j * 128, 128)],
                            s_sh.at[srcb], add=True)
            return carry2
        lax.fori_loop(0, _SUBS, _sub, None)
        return carry
    lax.fori_loop(0, _NSUP, _s_super, None)
    plsc.subcore_barrier()

    @pl.when(c == 0)
    def _():
        pltpu.sync_copy(s_sh.at[pl.ds(sid * _SW, _SW)],
                        s_hbm.at[pl.ds(sid * _SW, _SW)])


_sc1_stage = functools.partial(
    pl.kernel,
    out_type=[
        jax.ShapeDtypeStruct((_NPAD,), jnp.float32),
        jax.ShapeDtypeStruct((_NPAD,), jnp.float32),
        jax.ShapeDtypeStruct((_EPAD,), jnp.float32),
    ],
    mesh=plsc.VectorSubcoreMesh(core_axis_name="c", subcore_axis_name="s",
                                num_cores=_NC, num_subcores=_NS),
    compiler_params=_SC_PARAMS,
    scratch_types=[
        pltpu.VMEM((_NPAD,), jnp.float32),
        pltpu.VMEM((_SUP,), jnp.int32),
        pltpu.VMEM((_SUP,), jnp.int32),
        pltpu.VMEM((_SUP,), jnp.float32),
        pltpu.VMEM((_SUP,), jnp.float32),
        pltpu.VMEM((_SUP,), jnp.float32),
        pltpu.VMEM((128,), jnp.int32),
        pltpu.VMEM((128,), jnp.int32),
        pltpu.VMEM((_SW,), jnp.float32),
        pltpu.VMEM_SHARED((_NPAD,), jnp.float32),
        pltpu.VMEM_SHARED((_NPAD,), jnp.float32),
    ],
)(_sc1_body)


def _lstm_body(x_ref, wih_ref, whhT_ref, b_ref, w1T_ref, out_ref):
    x = x_ref[...]
    wih = wih_ref[...]
    b = b_ref[...]
    whhT = whhT_ref[...].astype(jnp.bfloat16)
    h = jnp.zeros((_BN, _H), jnp.float32)
    c = jnp.zeros((_BN, _H), jnp.float32)
    for t in range(_T):
        xt = x[:, t:t + 1]
        g = xt * wih + b
        g = g + jnp.dot(h.astype(jnp.bfloat16), whhT,
                        preferred_element_type=jnp.float32)
        i = jax.nn.sigmoid(g[:, 0:_H])
        f = jax.nn.sigmoid(g[:, _H:2 * _H])
        gg = jnp.tanh(g[:, 2 * _H:3 * _H])
        o = jax.nn.sigmoid(g[:, 3 * _H:4 * _H])
        c = f * c + i * gg
        h = o * jnp.tanh(c)
    xw = jnp.dot(jax.nn.relu(h), w1T_ref[...],
                 preferred_element_type=jnp.float32)
    out_ref[0, :, :] = xw[:, 0:32]
    out_ref[1, :, :] = xw[:, 32:64]


def _lstm_stage(xpad, wih, whhT, b, w1T):
    return pl.pallas_call(
        _lstm_body,
        grid=(_GA,),
        in_specs=[
            pl.BlockSpec((_BN, _T), lambda i: (i, 0)),
            pl.BlockSpec((1, 4 * _H), lambda i: (0, 0)),
            pl.BlockSpec((_H, 4 * _H), lambda i: (0, 0)),
            pl.BlockSpec((1, 4 * _H), lambda i: (0, 0)),
            pl.BlockSpec((_H, _H), lambda i: (0, 0)),
        ],
        out_specs=pl.BlockSpec((2, _BN, 32), lambda i: (0, i, 0)),
        out_shape=jax.ShapeDtypeStruct((2, _NPAD, 32), jnp.float32),
    )(xpad, wih, whhT, b, w1T)


def _sc2_body(src_hbm, dst_hbm, w_hbm, xw1_hbm,
              y1_hbm,
              src_v, dst_v, w_v, rows_a, rows_b, dstb_a, dstb_b,
              y1_sh, gsem, ssem):
    c = lax.axis_index("c")
    sid = lax.axis_index("s")
    ebase = sid * _EPT
    z16 = jnp.zeros((16,), jnp.float32)
    bufs = (rows_a, rows_b)
    dbufs = (dstb_a, dstb_b)
    xw1_page = xw1_hbm.at[c]
    y1_page = y1_hbm.at[c]

    def _zrows(e, carry):
        rows_a[e, pl.ds(0, 16)] = z16
        rows_a[e, pl.ds(16, 16)] = z16
        return carry
    lax.fori_loop(0, 128, _zrows, None)

    def _zy(k, carry):
        pltpu.sync_copy(rows_a, y1_sh.at[pl.ds(sid * _SW + k * 128, 128)])
        return carry
    lax.fori_loop(0, _SW // 128, _zy, None)
    pltpu.sync_copy(rows_a.at[pl.ds(0, 64)],
                    y1_sh.at[pl.ds(sid * _SW + (_SW // 128) * 128, 64)])
    plsc.subcore_barrier()

    def _main_super(ks, carry):
        base = ebase + ks * _SUP2
        pltpu.sync_copy(src_hbm.at[pl.ds(base, _SUP2)], src_v)
        pltpu.sync_copy(dst_hbm.at[pl.ds(base, _SUP2)], dst_v)
        pltpu.sync_copy(w_hbm.at[pl.ds(base, _SUP2)], w_v)

        sdesc = [None, None]
        gdesc = [None, None]
        gdesc[0] = pltpu.async_copy(
            xw1_page.at[src_v.at[pl.ds(0, 128)]], rows_a, gsem)
        for j in range(_SUBS2):
            p = j & 1
            q = 1 - p
            if j + 1 < _SUBS2:
                if sdesc[q] is not None:
                    sdesc[q].wait()
                    sdesc[q] = None
                gdesc[q] = pltpu.async_copy(
                    xw1_page.at[src_v.at[pl.ds((j + 1) * 128, 128)]],
                    bufs[q], gsem)
            gdesc[p].wait()

            def _scale(g, carry3, _j=j, _p=p):
                w16 = w_v[pl.ds(_j * 128 + g * 16, 16)]
                for k in range(16):
                    wk = w16[k]
                    r = bufs[_p]
                    e = g * 16 + k
                    r[e, pl.ds(0, 16)] = r[e, pl.ds(0, 16)] * wk
                    r[e, pl.ds(16, 16)] = r[e, pl.ds(16, 16)] * wk
                return carry3
            lax.fori_loop(0, 8, _scale, None)

            def _cp(g, carry3, _j=j, _p=p):
                dbufs[_p][pl.ds(g * 16, 16)] = \
                    dst_v[pl.ds(_j * 128 + g * 16, 16)]
                return carry3
            lax.fori_loop(0, 8, _cp, None)
            sdesc[p] = pltpu.async_copy(bufs[p], y1_sh.at[dbufs[p]],
                                        ssem, add=True)
        for d in sdesc:
            if d is not None:
                d.wait()
        return carry
    lax.fori_loop(0, _NSUP2, _main_super, None)
    plsc.subcore_barrier()

    def _wb(k, carry):
        pltpu.sync_copy(y1_sh.at[pl.ds(sid * _SW + k * 1024, 1024)],
                        y1_page.at[pl.ds(sid * _SW + k * 1024, 1024)])
        return carry
    lax.fori_loop(0, 3, _wb, None)
    pltpu.sync_copy(y1_sh.at[pl.ds(sid * _SW + 3072, 64)],
                    y1_page.at[pl.ds(sid * _SW + 3072, 64)])


_sc2_stage = functools.partial(
    pl.kernel,
    out_type=[
        jax.ShapeDtypeStruct((2, _NPAD, 32), jnp.float32),
    ],
    mesh=plsc.VectorSubcoreMesh(core_axis_name="c", subcore_axis_name="s",
                                num_cores=_NC, num_subcores=_NS),
    compiler_params=_SC_PARAMS,
    scratch_types=[
        pltpu.VMEM((_SUP2,), jnp.int32),
        pltpu.VMEM((_SUP2,), jnp.int32),
        pltpu.VMEM((_SUP2,), jnp.float32),
        pltpu.VMEM((128, 32), jnp.float32),
        pltpu.VMEM((128, 32), jnp.float32),
        pltpu.VMEM((128,), jnp.int32),
        pltpu.VMEM((128,), jnp.int32),
        pltpu.VMEM_SHARED((_NPAD, 32), jnp.float32),
        pltpu.SemaphoreType.DMA,
        pltpu.SemaphoreType.DMA,
    ],
)(_sc2_body)


def _final_body(y1_ref, xw1_ref, dis_ref, s_ref, b1_ref, w2T_ref, b2_ref,
                out_ref, acc_ref):
    i = pl.program_id(0)

    @pl.when(i == 0)
    def _():
        acc_ref[...] = jnp.zeros_like(acc_ref)

    dis = dis_ref[...]
    sv = s_ref[...]
    rowid = i * _BN + lax.broadcasted_iota(jnp.int32, (_BN, 1), 0)
    mask = rowid < _N
    a = jnp.where(mask, dis * (sv + dis), 0.0)
    y1 = jnp.concatenate([y1_ref[0], y1_ref[1]], axis=1)
    xw1 = jnp.concatenate([xw1_ref[0], xw1_ref[1]], axis=1)
    z = jax.nn.relu(dis * (y1 + dis * xw1) + b1_ref[...])
    z = jnp.where(mask, z, 0.0)
    acc_ref[...] += jnp.sum(z * a, axis=0, keepdims=True)

    @pl.when(i == _GA - 1)
    def _():
        out_ref[...] = (jnp.dot(acc_ref[...], w2T_ref[...],
                                preferred_element_type=jnp.float32)
                        * np.float32(1.0 / _N) + b2_ref[...])


def _final_stage(y1p, xw1p, dis, s, b1, w2T, b2):
    return pl.pallas_call(
        _final_body,
        grid=(_GA,),
        in_specs=[
            pl.BlockSpec((2, _BN, 32), lambda i: (0, i, 0)),
            pl.BlockSpec((2, _BN, 32), lambda i: (0, i, 0)),
            pl.BlockSpec((_BN, 1), lambda i: (i, 0)),
            pl.BlockSpec((_BN, 1), lambda i: (i, 0)),
            pl.BlockSpec((1, _H), lambda i: (0, 0)),
            pl.BlockSpec((_H, _OUT), lambda i: (0, 0)),
            pl.BlockSpec((1, _OUT), lambda i: (0, 0)),
        ],
        out_specs=pl.BlockSpec((1, _OUT), lambda i: (0, 0)),
        out_shape=jax.ShapeDtypeStruct((1, _OUT), jnp.float32),
        scratch_shapes=[pltpu.VMEM((1, _OUT), jnp.float32)],
    )(y1p, xw1p, dis, s, b1, w2T, b2)


def kernel(node_features, edge_index, edge_attributes,
           W_ih, W_hh, b_ih, b_hh, W1, b1, W2, b2):
    f32 = jnp.float32
    xpad = jnp.zeros((_NPAD, _T), f32).at[:_N].set(node_features)
    wih = W_ih[:, 0][None, :]
    whhT = W_hh.T
    bsum = (b_ih + b_hh)[None, :]
    w1T = W1.T

    epad = _EPAD - _E
    zi = jnp.zeros((epad,), jnp.int32)
    srcp = jnp.concatenate([edge_index[0], zi])
    dstp = jnp.concatenate([edge_index[1], zi])
    wp = jnp.concatenate([edge_attributes, jnp.zeros((epad,), f32)])

    xw1p = _lstm_stage(xpad, wih, whhT, bsum, w1T)

    dis, s, wsc = _sc1_stage(srcp, dstp, wp)

    (y1p,) = _sc2_stage(srcp, dstp, wsc, xw1p)

    return _final_stage(y1p, xw1p, dis[:, None], s[:, None],
                        b1[None, :], W2.T, b2[None, :])


# speedup vs baseline: 1.2213x; 1.0724x over previous
import functools

import jax
import jax.numpy as jnp
import numpy as np
from jax import lax
from jax.experimental import pallas as pl
from jax.experimental.pallas import tpu as pltpu
from jax.experimental.pallas import tpu_sc as plsc

_N = 50000
_T = 16
_H = 64
_OUT = 64

_BN = 512
_GA = 98
_NPAD = _BN * _GA
_SW = _NPAD // 16

_E = 800000
_EPT = 50176
_EPAD = 16 * _EPT
_SUP = 3584
_NSUP = _EPT // _SUP
_SUBS = _SUP // 128
_G16 = _SUP // 16

_SUP2 = 1792
_NSUP2 = _EPT // _SUP2
_SUBS2 = _SUP2 // 128
_G162 = _SUP2 // 16

_NC = 2
_NS = 16

_SC_PARAMS = pltpu.CompilerParams(needs_layout_passes=False,
                                  use_tc_tiling_on_sc=False)


def _sc1_body(src_hbm, dst_hbm, w_hbm,
              dis_hbm, s_hbm, wsc_hbm,
              dis_v, src_v, dst_v, w_v, val_v, wsc_v, dstb, srcb, stripe_v,
              deg_sh, s_sh):
    c = lax.axis_index("c")
    sid = lax.axis_index("s")
    ebase = sid * _EPT
    z16 = jnp.zeros((16,), jnp.float32)

    def _zstripe(i, carry):
        stripe_v[pl.ds(i * 16, 16)] = z16
        return carry
    lax.fori_loop(0, _SW // 16, _zstripe, None)
    pltpu.sync_copy(stripe_v, deg_sh.at[pl.ds(sid * _SW, _SW)])
    pltpu.sync_copy(stripe_v, s_sh.at[pl.ds(sid * _SW, _SW)])
    plsc.subcore_barrier()

    def _deg_super(ks, carry):
        base = ebase + ks * _SUP
        pltpu.sync_copy(dst_hbm.at[pl.ds(base, _SUP)], dst_v)
        pltpu.sync_copy(w_hbm.at[pl.ds(base, _SUP)], w_v)

        def _sub(j, carry2):
            def _cp(g, carry3):
                dstb[pl.ds(g * 16, 16)] = dst_v[pl.ds(j * 128 + g * 16, 16)]
                return carry3
            lax.fori_loop(0, 8, _cp, None)
            pltpu.sync_copy(w_v.at[pl.ds(j * 128, 128)],
                            deg_sh.at[dstb], add=True)
            return carry2
        lax.fori_loop(0, _SUBS, _sub, None)
        return carry
    lax.fori_loop(0, _NSUP, _deg_super, None)
    plsc.subcore_barrier()

    pltpu.sync_copy(deg_sh.at[pl.ds(sid * _SW, _SW)], stripe_v)

    def _newton(i, carry):
        v = stripe_v[pl.ds(i * 16, 16)] + 1.0
        iv = lax.bitcast_convert_type(v, jnp.int32)
        yi = jnp.int32(0x5F3759DF) - lax.shift_right_logical(iv, 1)
        y = lax.bitcast_convert_type(yi, jnp.float32)
        y = y * (1.5 - 0.5 * v * y * y)
        y = y * (1.5 - 0.5 * v * y * y)
        y = y * (1.5 - 0.5 * v * y * y)
        y = y * (1.5 - 0.5 * v * y * y)
        stripe_v[pl.ds(i * 16, 16)] = y
        return carry
    lax.fori_loop(0, _SW // 16, _newton, None)
    pltpu.sync_copy(stripe_v, deg_sh.at[pl.ds(sid * _SW, _SW)])

    @pl.when(c == 0)
    def _():
        pltpu.sync_copy(stripe_v, dis_hbm.at[pl.ds(sid * _SW, _SW)])
    plsc.subcore_barrier()

    pltpu.sync_copy(deg_sh, dis_v)

    def _s_super(ks, carry):
        base = ebase + ks * _SUP
        pltpu.sync_copy(src_hbm.at[pl.ds(base, _SUP)], src_v)
        pltpu.sync_copy(dst_hbm.at[pl.ds(base, _SUP)], dst_v)
        pltpu.sync_copy(w_hbm.at[pl.ds(base, _SUP)], w_v)

        def _val(g, carry2):
            d16 = dst_v[pl.ds(g * 16, 16)]
            s16 = src_v[pl.ds(g * 16, 16)]
            w16 = w_v[pl.ds(g * 16, 16)]
            disd = plsc.load_gather(dis_v, [d16])
            diss = plsc.load_gather(dis_v, [s16])
            val_v[pl.ds(g * 16, 16)] = w16 * disd
            wsc_v[pl.ds(g * 16, 16)] = w16 * diss
            return carry2
        lax.fori_loop(0, _G16, _val, None)

        @pl.when(c == 0)
        def _():
            pltpu.sync_copy(wsc_v, wsc_hbm.at[pl.ds(base, _SUP)])

        def _sub(j, carry2):
            def _cp(g, carry3):
                srcb[pl.ds(g * 16, 16)] = src_v[pl.ds(j * 128 + g * 16, 16)]
                return carry3
            lax.fori_loop(0, 8, _cp, None)
            pltpu.sync_copy(val_v.at[pl.ds(j * 128, 128)],
                            s_sh.at[srcb], add=True)
            return carry2
        lax.fori_loop(0, _SUBS, _sub, None)
        return carry
    lax.fori_loop(0, _NSUP, _s_super, None)
    plsc.subcore_barrier()

    @pl.when(c == 0)
    def _():
        pltpu.sync_copy(s_sh.at[pl.ds(sid * _SW, _SW)],
                        s_hbm.at[pl.ds(sid * _SW, _SW)])


_sc1_stage = functools.partial(
    pl.kernel,
    out_type=[
        jax.ShapeDtypeStruct((_NPAD,), jnp.float32),
        jax.ShapeDtypeStruct((_NPAD,), jnp.float32),
        jax.ShapeDtypeStruct((_EPAD,), jnp.float32),
    ],
    mesh=plsc.VectorSubcoreMesh(core_axis_name="c", subcore_axis_name="s",
                                num_cores=_NC, num_subcores=_NS),
    compiler_params=_SC_PARAMS,
    scratch_types=[
        pltpu.VMEM((_NPAD,), jnp.float32),
        pltpu.VMEM((_SUP,), jnp.int32),
        pltpu.VMEM((_SUP,), jnp.int32),
        pltpu.VMEM((_SUP,), jnp.float32),
        pltpu.VMEM((_SUP,), jnp.float32),
        pltpu.VMEM((_SUP,), jnp.float32),
        pltpu.VMEM((128,), jnp.int32),
        pltpu.VMEM((128,), jnp.int32),
        pltpu.VMEM((_SW,), jnp.float32),
        pltpu.VMEM_SHARED((_NPAD,), jnp.float32),
        pltpu.VMEM_SHARED((_NPAD,), jnp.float32),
    ],
)(_sc1_body)


def _lstm_body(x_ref, wih_ref, whhT_ref, b_ref, w1T_ref, out_ref):
    x = x_ref[...]
    wih = wih_ref[...]
    b = b_ref[...]
    whhT = whhT_ref[...].astype(jnp.bfloat16)
    h = jnp.zeros((_BN, _H), jnp.float32)
    c = jnp.zeros((_BN, _H), jnp.float32)
    for t in range(_T):
        xt = x[:, t:t + 1]
        g = xt * wih + b
        g = g + jnp.dot(h.astype(jnp.bfloat16), whhT,
                        preferred_element_type=jnp.float32)
        i = jax.nn.sigmoid(g[:, 0:_H])
        f = jax.nn.sigmoid(g[:, _H:2 * _H])
        gg = jnp.tanh(g[:, 2 * _H:3 * _H])
        o = jax.nn.sigmoid(g[:, 3 * _H:4 * _H])
        c = f * c + i * gg
        h = o * jnp.tanh(c)
    xw = jnp.dot(jax.nn.relu(h), w1T_ref[...],
                 preferred_element_type=jnp.float32)
    out_ref[0, :, :] = xw[:, 0:32]
    out_ref[1, :, :] = xw[:, 32:64]


def _lstm_stage(xpad, wih, whhT, b, w1T):
    return pl.pallas_call(
        _lstm_body,
        grid=(_GA,),
        in_specs=[
            pl.BlockSpec((_BN, _T), lambda i: (i, 0)),
            pl.BlockSpec((1, 4 * _H), lambda i: (0, 0)),
            pl.BlockSpec((_H, 4 * _H), lambda i: (0, 0)),
            pl.BlockSpec((1, 4 * _H), lambda i: (0, 0)),
            pl.BlockSpec((_H, _H), lambda i: (0, 0)),
        ],
        out_specs=pl.BlockSpec((2, _BN, 32), lambda i: (0, i, 0)),
        out_shape=jax.ShapeDtypeStruct((2, _NPAD, 32), jnp.float32),
    )(xpad, wih, whhT, b, w1T)


def _sc2_body(src_hbm, dst_hbm, w_hbm, xw1_hbm, dis_hbm, s_hbm, b1_hbm,
              vp_hbm,
              src_v, dst_v, w_v, rows_a, rows_b, dstb_a, dstb_b, pv,
              y1_sh, gsem, ssem):
    c = lax.axis_index("c")
    sid = lax.axis_index("s")
    ebase = sid * _EPT
    z16 = jnp.zeros((16,), jnp.float32)
    bufs = (rows_a, rows_b)
    dbufs = (dstb_a, dstb_b)
    xw1_page = xw1_hbm.at[c]

    def _zrows(e, carry):
        rows_a[e, pl.ds(0, 16)] = z16
        rows_a[e, pl.ds(16, 16)] = z16
        return carry
    lax.fori_loop(0, 128, _zrows, None)

    def _zy(k, carry):
        pltpu.sync_copy(rows_a, y1_sh.at[pl.ds(sid * _SW + k * 128, 128)])
        return carry
    lax.fori_loop(0, _SW // 128, _zy, None)
    pltpu.sync_copy(rows_a.at[pl.ds(0, 64)],
                    y1_sh.at[pl.ds(sid * _SW + (_SW // 128) * 128, 64)])
    plsc.subcore_barrier()

    def _main_super(ks, carry):
        base = ebase + ks * _SUP2
        pltpu.sync_copy(src_hbm.at[pl.ds(base, _SUP2)], src_v)
        pltpu.sync_copy(dst_hbm.at[pl.ds(base, _SUP2)], dst_v)
        pltpu.sync_copy(w_hbm.at[pl.ds(base, _SUP2)], w_v)

        sdesc = [None, None]
        gdesc = [None, None]
        gdesc[0] = pltpu.async_copy(
            xw1_page.at[src_v.at[pl.ds(0, 128)]], rows_a, gsem)
        for j in range(_SUBS2):
            p = j & 1
            q = 1 - p
            if j + 1 < _SUBS2:
                if sdesc[q] is not None:
                    sdesc[q].wait()
                    sdesc[q] = None
                gdesc[q] = pltpu.async_copy(
                    xw1_page.at[src_v.at[pl.ds((j + 1) * 128, 128)]],
                    bufs[q], gsem)
            gdesc[p].wait()

            def _scale(g, carry3, _j=j, _p=p):
                w16 = w_v[pl.ds(_j * 128 + g * 16, 16)]
                for k in range(16):
                    wk = w16[k]
                    r = bufs[_p]
                    e = g * 16 + k
                    r[e, pl.ds(0, 16)] = r[e, pl.ds(0, 16)] * wk
                    r[e, pl.ds(16, 16)] = r[e, pl.ds(16, 16)] * wk
                return carry3
            lax.fori_loop(0, 8, _scale, None)

            def _cp(g, carry3, _j=j, _p=p):
                dbufs[_p][pl.ds(g * 16, 16)] = \
                    dst_v[pl.ds(_j * 128 + g * 16, 16)]
                return carry3
            lax.fori_loop(0, 8, _cp, None)
            sdesc[p] = pltpu.async_copy(bufs[p], y1_sh.at[dbufs[p]],
                                        ssem, add=True)
        for d in sdesc:
            if d is not None:
                d.wait()
        return carry
    lax.fori_loop(0, _NSUP2, _main_super, None)
    plsc.subcore_barrier()

    nbase = sid * _SW
    pltpu.sync_copy(b1_hbm.at[c], pv)
    b1a = pv[pl.ds(0, 16)]
    b1b = pv[pl.ds(16, 16)]

    def _red(k, acc):
        rb = nbase + k * 112
        pltpu.sync_copy(y1_sh.at[pl.ds(rb, 112)], rows_a.at[pl.ds(0, 112)])
        pltpu.sync_copy(xw1_page.at[pl.ds(rb, 112)],
                        rows_b.at[pl.ds(0, 112)])
        pltpu.sync_copy(dis_hbm.at[pl.ds(rb, 112)], w_v.at[pl.ds(0, 112)])
        pltpu.sync_copy(s_hbm.at[pl.ds(rb, 112)], w_v.at[pl.ds(112, 112)])

        def _grp(g, acc2):
            a0, a1 = acc2
            d16 = w_v[pl.ds(g * 16, 16)]
            sv16 = w_v[pl.ds(112 + g * 16, 16)]
            ids = rb + g * 16 + lax.iota(jnp.int32, 16)
            a16 = jnp.where(ids < _N, d16 * (sv16 + d16), 0.0)
            for kk in range(16):
                e = g * 16 + kk
                dk = d16[kk]
                ak = a16[kk]
                y0 = rows_a[e, pl.ds(0, 16)]
                y1v = rows_a[e, pl.ds(16, 16)]
                x0 = rows_b[e, pl.ds(0, 16)]
                x1 = rows_b[e, pl.ds(16, 16)]
                z0 = jnp.maximum(dk * (y0 + dk * x0) + b1a, 0.0)
                z1 = jnp.maximum(dk * (y1v + dk * x1) + b1b, 0.0)
                a0 = a0 + ak * z0
                a1 = a1 + ak * z1
            return (a0, a1)
        return lax.fori_loop(0, 7, _grp, acc)
    acc0, acc1 = lax.fori_loop(0, _SW // 112, _red, (z16, z16))
    pv[pl.ds(0, 16)] = acc0
    pv[pl.ds(16, 16)] = acc1
    pltpu.sync_copy(pv, vp_hbm.at[c].at[sid])


_sc2_stage = functools.partial(
    pl.kernel,
    out_type=[
        jax.ShapeDtypeStruct((2, 16, 32), jnp.float32),
    ],
    mesh=plsc.VectorSubcoreMesh(core_axis_name="c", subcore_axis_name="s",
                                num_cores=_NC, num_subcores=_NS),
    compiler_params=_SC_PARAMS,
    scratch_types=[
        pltpu.VMEM((_SUP2,), jnp.int32),
        pltpu.VMEM((_SUP2,), jnp.int32),
        pltpu.VMEM((_SUP2,), jnp.float32),
        pltpu.VMEM((128, 32), jnp.float32),
        pltpu.VMEM((128, 32), jnp.float32),
        pltpu.VMEM((128,), jnp.int32),
        pltpu.VMEM((128,), jnp.int32),
        pltpu.VMEM((32,), jnp.float32),
        pltpu.VMEM_SHARED((_NPAD, 32), jnp.float32),
        pltpu.SemaphoreType.DMA,
        pltpu.SemaphoreType.DMA,
    ],
)(_sc2_body)


def _final_body(vp_ref, w2T_ref, b2_ref, out_ref):
    v0 = jnp.sum(vp_ref[0], axis=0, keepdims=True)
    v1 = jnp.sum(vp_ref[1], axis=0, keepdims=True)
    v = jnp.concatenate([v0, v1], axis=1)
    out_ref[...] = (jnp.dot(v, w2T_ref[...],
                            preferred_element_type=jnp.float32)
                    * np.float32(1.0 / _N) + b2_ref[...])


def _final_stage(vp, w2T, b2):
    return pl.pallas_call(
        _final_body,
        grid=(1,),
        in_specs=[
            pl.BlockSpec((2, 16, 32), lambda i: (0, 0, 0)),
            pl.BlockSpec((_H, _OUT), lambda i: (0, 0)),
            pl.BlockSpec((1, _OUT), lambda i: (0, 0)),
        ],
        out_specs=pl.BlockSpec((1, _OUT), lambda i: (0, 0)),
        out_shape=jax.ShapeDtypeStruct((1, _OUT), jnp.float32),
    )(vp, w2T, b2)


def kernel(node_features, edge_index, edge_attributes,
           W_ih, W_hh, b_ih, b_hh, W1, b1, W2, b2):
    f32 = jnp.float32
    xpad = jnp.zeros((_NPAD, _T), f32).at[:_N].set(node_features)
    wih = W_ih[:, 0][None, :]
    whhT = W_hh.T
    bsum = (b_ih + b_hh)[None, :]
    w1T = W1.T

    epad = _EPAD - _E
    zi = jnp.zeros((epad,), jnp.int32)
    srcp = jnp.concatenate([edge_index[0], zi])
    dstp = jnp.concatenate([edge_index[1], zi])
    wp = jnp.concatenate([edge_attributes, jnp.zeros((epad,), f32)])

    xw1p = _lstm_stage(xpad, wih, whhT, bsum, w1T)

    dis, s, wsc = _sc1_stage(srcp, dstp, wp)

    (vp,) = _sc2_stage(srcp, dstp, wsc, xw1p, dis, s, b1.reshape(2, 32))

    return _final_stage(vp, W2.T, b2[None, :])


# speedup vs baseline: 1.2758x; 1.0446x over previous
import functools

import jax
import jax.numpy as jnp
import numpy as np
from jax import lax
from jax.experimental import pallas as pl
from jax.experimental.pallas import tpu as pltpu
from jax.experimental.pallas import tpu_sc as plsc

_N = 50000
_T = 16
_H = 64
_OUT = 64

_BN = 512
_GA = 98
_NPAD = _BN * _GA
_SW = _NPAD // 16

_E = 800000
_EPT = 50176
_EPAD = 16 * _EPT
_SUP = 3584
_NSUP = _EPT // _SUP
_SUBS = _SUP // 128
_G16 = _SUP // 16

_SUP2 = 1792
_NSUP2 = _EPT // _SUP2
_SUBS2 = _SUP2 // 128
_G162 = _SUP2 // 16

_NC = 2
_NS = 16

_SC_PARAMS = pltpu.CompilerParams(needs_layout_passes=False,
                                  use_tc_tiling_on_sc=False)


def _sc1_body(src_hbm, dst_hbm, w_hbm,
              dis_hbm, s_hbm, wsc_hbm,
              dis_v, src_v, dst_v, w_v, val_v, wsc_v, dstb, srcb, stripe_v,
              deg_sh, s_sh):
    c = lax.axis_index("c")
    sid = lax.axis_index("s")
    ebase = sid * _EPT
    z16 = jnp.zeros((16,), jnp.float32)

    def _zstripe(i, carry):
        stripe_v[pl.ds(i * 16, 16)] = z16
        return carry
    lax.fori_loop(0, _SW // 16, _zstripe, None)
    pltpu.sync_copy(stripe_v, deg_sh.at[pl.ds(sid * _SW, _SW)])
    pltpu.sync_copy(stripe_v, s_sh.at[pl.ds(sid * _SW, _SW)])
    plsc.subcore_barrier()

    def _deg_super(ks, carry):
        base = ebase + ks * _SUP
        pltpu.sync_copy(dst_hbm.at[pl.ds(base, _SUP)], dst_v)
        pltpu.sync_copy(w_hbm.at[pl.ds(base, _SUP)], w_v)

        def _sub(j, carry2):
            def _cp(g, carry3):
                dstb[pl.ds(g * 16, 16)] = dst_v[pl.ds(j * 128 + g * 16, 16)]
                return carry3
            lax.fori_loop(0, 8, _cp, None)
            pltpu.sync_copy(w_v.at[pl.ds(j * 128, 128)],
                            deg_sh.at[dstb], add=True)
            return carry2
        lax.fori_loop(0, _SUBS, _sub, None)
        return carry
    lax.fori_loop(0, _NSUP, _deg_super, None)
    plsc.subcore_barrier()

    pltpu.sync_copy(deg_sh.at[pl.ds(sid * _SW, _SW)], stripe_v)

    def _newton(i, carry):
        v = stripe_v[pl.ds(i * 16, 16)] + 1.0
        iv = lax.bitcast_convert_type(v, jnp.int32)
        yi = jnp.int32(0x5F3759DF) - lax.shift_right_logical(iv, 1)
        y = lax.bitcast_convert_type(yi, jnp.float32)
        y = y * (1.5 - 0.5 * v * y * y)
        y = y * (1.5 - 0.5 * v * y * y)
        y = y * (1.5 - 0.5 * v * y * y)
        y = y * (1.5 - 0.5 * v * y * y)
        stripe_v[pl.ds(i * 16, 16)] = y
        return carry
    lax.fori_loop(0, _SW // 16, _newton, None)
    pltpu.sync_copy(stripe_v, deg_sh.at[pl.ds(sid * _SW, _SW)])

    @pl.when(c == 0)
    def _():
        pltpu.sync_copy(stripe_v, dis_hbm.at[pl.ds(sid * _SW, _SW)])
    plsc.subcore_barrier()

    pltpu.sync_copy(deg_sh, dis_v)

    def _s_super(ks, carry):
        base = ebase + ks * _SUP
        pltpu.sync_copy(src_hbm.at[pl.ds(base, _SUP)], src_v)
        pltpu.sync_copy(dst_hbm.at[pl.ds(base, _SUP)], dst_v)
        pltpu.sync_copy(w_hbm.at[pl.ds(base, _SUP)], w_v)

        def _val(g, carry2):
            d16 = dst_v[pl.ds(g * 16, 16)]
            s16 = src_v[pl.ds(g * 16, 16)]
            w16 = w_v[pl.ds(g * 16, 16)]
            disd = plsc.load_gather(dis_v, [d16])
            diss = plsc.load_gather(dis_v, [s16])
            val_v[pl.ds(g * 16, 16)] = w16 * disd
            wsc_v[pl.ds(g * 16, 16)] = w16 * diss
            return carry2
        lax.fori_loop(0, _G16, _val, None)

        @pl.when(c == 0)
        def _():
            pltpu.sync_copy(wsc_v, wsc_hbm.at[pl.ds(base, _SUP)])

        def _sub(j, carry2):
            def _cp(g, carry3):
                srcb[pl.ds(g * 16, 16)] = src_v[pl.ds(j * 128 + g * 16, 16)]
                return carry3
            lax.fori_loop(0, 8, _cp, None)
            pltpu.sync_copy(val_v.at[pl.ds(j * 128, 128)],
                            s_sh.at[srcb], add=True)
            return carry2
        lax.fori_loop(0, _SUBS, _sub, None)
        return carry
    lax.fori_loop(0, _NSUP, _s_super, None)
    plsc.subcore_barrier()

    @pl.when(c == 0)
    def _():
        pltpu.sync_copy(s_sh.at[pl.ds(sid * _SW, _SW)],
                        s_hbm.at[pl.ds(sid * _SW, _SW)])


_sc1_stage = functools.partial(
    pl.kernel,
    out_type=[
        jax.ShapeDtypeStruct((_NPAD,), jnp.float32),
        jax.ShapeDtypeStruct((_NPAD,), jnp.float32),
        jax.ShapeDtypeStruct((_EPAD,), jnp.float32),
    ],
    mesh=plsc.VectorSubcoreMesh(core_axis_name="c", subcore_axis_name="s",
                                num_cores=_NC, num_subcores=_NS),
    compiler_params=_SC_PARAMS,
    scratch_types=[
        pltpu.VMEM((_NPAD,), jnp.float32),
        pltpu.VMEM((_SUP,), jnp.int32),
        pltpu.VMEM((_SUP,), jnp.int32),
        pltpu.VMEM((_SUP,), jnp.float32),
        pltpu.VMEM((_SUP,), jnp.float32),
        pltpu.VMEM((_SUP,), jnp.float32),
        pltpu.VMEM((128,), jnp.int32),
        pltpu.VMEM((128,), jnp.int32),
        pltpu.VMEM((_SW,), jnp.float32),
        pltpu.VMEM_SHARED((_NPAD,), jnp.float32),
        pltpu.VMEM_SHARED((_NPAD,), jnp.float32),
    ],
)(_sc1_body)


def _lstm_body(x_ref, wih_ref, whhT_ref, b_ref, w1T_ref, out_ref):
    x = x_ref[...]
    wih = wih_ref[...]
    b = b_ref[...]
    whhT = whhT_ref[...].astype(jnp.bfloat16)
    h = jnp.zeros((_BN, _H), jnp.float32)
    c = jnp.zeros((_BN, _H), jnp.float32)
    for t in range(_T):
        xt = x[:, t:t + 1]
        g = xt * wih + b
        g = g + jnp.dot(h.astype(jnp.bfloat16), whhT,
                        preferred_element_type=jnp.float32)
        i = 0.5 + 0.5 * jnp.tanh(0.5 * g[:, 0:_H])
        f = 0.5 + 0.5 * jnp.tanh(0.5 * g[:, _H:2 * _H])
        gg = jnp.tanh(g[:, 2 * _H:3 * _H])
        o = 0.5 + 0.5 * jnp.tanh(0.5 * g[:, 3 * _H:4 * _H])
        c = f * c + i * gg
        h = o * jnp.tanh(c)
    xw = jnp.dot(jax.nn.relu(h), w1T_ref[...],
                 preferred_element_type=jnp.float32)
    out_ref[0, :, :] = xw[:, 0:32]
    out_ref[1, :, :] = xw[:, 32:64]


def _lstm_stage(xpad, wih, whhT, b, w1T):
    return pl.pallas_call(
        _lstm_body,
        grid=(_GA,),
        in_specs=[
            pl.BlockSpec((_BN, _T), lambda i: (i, 0)),
            pl.BlockSpec((1, 4 * _H), lambda i: (0, 0)),
            pl.BlockSpec((_H, 4 * _H), lambda i: (0, 0)),
            pl.BlockSpec((1, 4 * _H), lambda i: (0, 0)),
            pl.BlockSpec((_H, _H), lambda i: (0, 0)),
        ],
        out_specs=pl.BlockSpec((2, _BN, 32), lambda i: (0, i, 0)),
        out_shape=jax.ShapeDtypeStruct((2, _NPAD, 32), jnp.float32),
    )(xpad, wih, whhT, b, w1T)


def _sc2_body(src_hbm, dst_hbm, w_hbm, xw1_hbm, dis_hbm, s_hbm, b1_hbm,
              vp_hbm,
              src_v, dst_v, w_v, rows_a, rows_b, dstb_a, dstb_b, pv,
              y1_sh, gsem, ssem):
    c = lax.axis_index("c")
    sid = lax.axis_index("s")
    ebase = sid * _EPT
    z16 = jnp.zeros((16,), jnp.float32)
    bufs = (rows_a, rows_b)
    dbufs = (dstb_a, dstb_b)
    xw1_page = xw1_hbm.at[c]

    def _zrows(e, carry):
        rows_a[e, pl.ds(0, 16)] = z16
        rows_a[e, pl.ds(16, 16)] = z16
        return carry
    lax.fori_loop(0, 128, _zrows, None)

    def _zy(k, carry):
        pltpu.sync_copy(rows_a, y1_sh.at[pl.ds(sid * _SW + k * 128, 128)])
        return carry
    lax.fori_loop(0, _SW // 128, _zy, None)
    pltpu.sync_copy(rows_a.at[pl.ds(0, 64)],
                    y1_sh.at[pl.ds(sid * _SW + (_SW // 128) * 128, 64)])
    plsc.subcore_barrier()

    def _main_super(ks, carry):
        base = ebase + ks * _SUP2
        pltpu.sync_copy(src_hbm.at[pl.ds(base, _SUP2)], src_v)
        pltpu.sync_copy(dst_hbm.at[pl.ds(base, _SUP2)], dst_v)
        pltpu.sync_copy(w_hbm.at[pl.ds(base, _SUP2)], w_v)

        sdesc = [None, None]
        gdesc = [None, None]
        gdesc[0] = pltpu.async_copy(
            xw1_page.at[src_v.at[pl.ds(0, 128)]], rows_a, gsem)
        for j in range(_SUBS2):
            p = j & 1
            q = 1 - p
            if j + 1 < _SUBS2:
                if sdesc[q] is not None:
                    sdesc[q].wait()
                    sdesc[q] = None
                gdesc[q] = pltpu.async_copy(
                    xw1_page.at[src_v.at[pl.ds((j + 1) * 128, 128)]],
                    bufs[q], gsem)
            gdesc[p].wait()

            def _scale(g, carry3, _j=j, _p=p):
                w16 = w_v[pl.ds(_j * 128 + g * 16, 16)]
                for k in range(16):
                    wk = w16[k]
                    r = bufs[_p]
                    e = g * 16 + k
                    r[e, pl.ds(0, 16)] = r[e, pl.ds(0, 16)] * wk
                    r[e, pl.ds(16, 16)] = r[e, pl.ds(16, 16)] * wk
                return carry3
            lax.fori_loop(0, 8, _scale, None)

            def _cp(g, carry3, _j=j, _p=p):
                dbufs[_p][pl.ds(g * 16, 16)] = \
                    dst_v[pl.ds(_j * 128 + g * 16, 16)]
                return carry3
            lax.fori_loop(0, 8, _cp, None)
            sdesc[p] = pltpu.async_copy(bufs[p], y1_sh.at[dbufs[p]],
                                        ssem, add=True)
        for d in sdesc:
            if d is not None:
                d.wait()
        return carry
    lax.fori_loop(0, _NSUP2, _main_super, None)
    plsc.subcore_barrier()

    nbase = sid * _SW
    pltpu.sync_copy(b1_hbm.at[c], pv)
    b1a = pv[pl.ds(0, 16)]
    b1b = pv[pl.ds(16, 16)]

    def _red(k, acc):
        rb = nbase + k * 112
        pltpu.sync_copy(y1_sh.at[pl.ds(rb, 112)], rows_a.at[pl.ds(0, 112)])
        pltpu.sync_copy(xw1_page.at[pl.ds(rb, 112)],
                        rows_b.at[pl.ds(0, 112)])
        pltpu.sync_copy(dis_hbm.at[pl.ds(rb, 112)], w_v.at[pl.ds(0, 112)])
        pltpu.sync_copy(s_hbm.at[pl.ds(rb, 112)], w_v.at[pl.ds(112, 112)])

        def _grp(g, acc2):
            a0, a1 = acc2
            d16 = w_v[pl.ds(g * 16, 16)]
            sv16 = w_v[pl.ds(112 + g * 16, 16)]
            ids = rb + g * 16 + lax.iota(jnp.int32, 16)
            a16 = jnp.where(ids < _N, d16 * (sv16 + d16), 0.0)
            for kk in range(16):
                e = g * 16 + kk
                dk = d16[kk]
                ak = a16[kk]
                y0 = rows_a[e, pl.ds(0, 16)]
                y1v = rows_a[e, pl.ds(16, 16)]
                x0 = rows_b[e, pl.ds(0, 16)]
                x1 = rows_b[e, pl.ds(16, 16)]
                z0 = jnp.maximum(dk * (y0 + dk * x0) + b1a, 0.0)
                z1 = jnp.maximum(dk * (y1v + dk * x1) + b1b, 0.0)
                a0 = a0 + ak * z0
                a1 = a1 + ak * z1
            return (a0, a1)
        return lax.fori_loop(0, 7, _grp, acc)
    acc0, acc1 = lax.fori_loop(0, _SW // 112, _red, (z16, z16))
    pv[pl.ds(0, 16)] = acc0
    pv[pl.ds(16, 16)] = acc1
    pltpu.sync_copy(pv, vp_hbm.at[c].at[sid])


_sc2_stage = functools.partial(
    pl.kernel,
    out_type=[
        jax.ShapeDtypeStruct((2, 16, 32), jnp.float32),
    ],
    mesh=plsc.VectorSubcoreMesh(core_axis_name="c", subcore_axis_name="s",
                                num_cores=_NC, num_subcores=_NS),
    compiler_params=_SC_PARAMS,
    scratch_types=[
        pltpu.VMEM((_SUP2,), jnp.int32),
        pltpu.VMEM((_SUP2,), jnp.int32),
        pltpu.VMEM((_SUP2,), jnp.float32),
        pltpu.VMEM((128, 32), jnp.float32),
        pltpu.VMEM((128, 32), jnp.float32),
        pltpu.VMEM((128,), jnp.int32),
        pltpu.VMEM((128,), jnp.int32),
        pltpu.VMEM((32,), jnp.float32),
        pltpu.VMEM_SHARED((_NPAD, 32), jnp.float32),
        pltpu.SemaphoreType.DMA,
        pltpu.SemaphoreType.DMA,
    ],
)(_sc2_body)


def _final_body(vp_ref, w2T_ref, b2_ref, out_ref):
    v0 = jnp.sum(vp_ref[0], axis=0, keepdims=True)
    v1 = jnp.sum(vp_ref[1], axis=0, keepdims=True)
    v = jnp.concatenate([v0, v1], axis=1)
    out_ref[...] = (jnp.dot(v, w2T_ref[...],
                            preferred_element_type=jnp.float32)
                    * np.float32(1.0 / _N) + b2_ref[...])


def _final_stage(vp, w2T, b2):
    return pl.pallas_call(
        _final_body,
        grid=(1,),
        in_specs=[
            pl.BlockSpec((2, 16, 32), lambda i: (0, 0, 0)),
            pl.BlockSpec((_H, _OUT), lambda i: (0, 0)),
            pl.BlockSpec((1, _OUT), lambda i: (0, 0)),
        ],
        out_specs=pl.BlockSpec((1, _OUT), lambda i: (0, 0)),
        out_shape=jax.ShapeDtypeStruct((1, _OUT), jnp.float32),
    )(vp, w2T, b2)


def kernel(node_features, edge_index, edge_attributes,
           W_ih, W_hh, b_ih, b_hh, W1, b1, W2, b2):
    f32 = jnp.float32
    xpad = jnp.zeros((_NPAD, _T), f32).at[:_N].set(node_features)
    wih = W_ih[:, 0][None, :]
    whhT = W_hh.T
    bsum = (b_ih + b_hh)[None, :]
    w1T = W1.T

    epad = _EPAD - _E
    zi = jnp.zeros((epad,), jnp.int32)
    srcp = jnp.concatenate([edge_index[0], zi])
    dstp = jnp.concatenate([edge_index[1], zi])
    wp = jnp.concatenate([edge_attributes, jnp.zeros((epad,), f32)])

    xw1p = _lstm_stage(xpad, wih, whhT, bsum, w1T)

    dis, s, wsc = _sc1_stage(srcp, dstp, wp)

    (vp,) = _sc2_stage(srcp, dstp, wsc, xw1p, dis, s, b1.reshape(2, 32))

    return _final_stage(vp, W2.T, b2[None, :])


# speedup vs baseline: 1.3101x; 1.0269x over previous
import functools

import jax
import jax.numpy as jnp
import numpy as np
from jax import lax
from jax.experimental import pallas as pl
from jax.experimental.pallas import tpu as pltpu
from jax.experimental.pallas import tpu_sc as plsc

_N = 50000
_T = 16
_H = 64
_OUT = 64

_BN = 512
_GA = 98
_NPAD = _BN * _GA
_SW = _NPAD // 16

_E = 800000
_EPT = 50176
_EPAD = 16 * _EPT
_SUP = 3584
_NSUP = _EPT // _SUP
_SUBS = _SUP // 128
_G16 = _SUP // 16

_SUP2 = 1792
_NSUP2 = _EPT // _SUP2
_SUBS2 = _SUP2 // 128
_G162 = _SUP2 // 16

_NC = 2
_NS = 16

_SC_PARAMS = pltpu.CompilerParams(needs_layout_passes=False,
                                  use_tc_tiling_on_sc=False)


def _sc1_body(src_hbm, dst_hbm, w_hbm,
              dis_hbm, s_hbm, wsc_hbm,
              dis_v, src_v, dst_v, w_v, val_v, wsc_v, dstb, srcb, stripe_v,
              deg_sh, s_sh):
    c = lax.axis_index("c")
    sid = lax.axis_index("s")
    ebase = sid * _EPT
    z16 = jnp.zeros((16,), jnp.float32)

    def _zstripe(i, carry):
        stripe_v[pl.ds(i * 16, 16)] = z16
        return carry
    lax.fori_loop(0, _SW // 16, _zstripe, None)
    pltpu.sync_copy(stripe_v, deg_sh.at[pl.ds(sid * _SW, _SW)])
    pltpu.sync_copy(stripe_v, s_sh.at[pl.ds(sid * _SW, _SW)])
    plsc.subcore_barrier()

    def _deg_super(ks, carry):
        base = ebase + ks * _SUP
        pltpu.sync_copy(dst_hbm.at[pl.ds(base, _SUP)], dst_v)
        pltpu.sync_copy(w_hbm.at[pl.ds(base, _SUP)], w_v)

        def _sub(j, carry2):
            def _cp(g, carry3):
                dstb[pl.ds(g * 16, 16)] = dst_v[pl.ds(j * 128 + g * 16, 16)]
                return carry3
            lax.fori_loop(0, 8, _cp, None)
            pltpu.sync_copy(w_v.at[pl.ds(j * 128, 128)],
                            deg_sh.at[dstb], add=True)
            return carry2
        lax.fori_loop(0, _SUBS, _sub, None)
        return carry
    lax.fori_loop(0, _NSUP, _deg_super, None)
    plsc.subcore_barrier()

    pltpu.sync_copy(deg_sh.at[pl.ds(sid * _SW, _SW)], stripe_v)

    def _newton(i, carry):
        v = stripe_v[pl.ds(i * 16, 16)] + 1.0
        iv = lax.bitcast_convert_type(v, jnp.int32)
        yi = jnp.int32(0x5F3759DF) - lax.shift_right_logical(iv, 1)
        y = lax.bitcast_convert_type(yi, jnp.float32)
        y = y * (1.5 - 0.5 * v * y * y)
        y = y * (1.5 - 0.5 * v * y * y)
        y = y * (1.5 - 0.5 * v * y * y)
        y = y * (1.5 - 0.5 * v * y * y)
        stripe_v[pl.ds(i * 16, 16)] = y
        return carry
    lax.fori_loop(0, _SW // 16, _newton, None)
    pltpu.sync_copy(stripe_v, deg_sh.at[pl.ds(sid * _SW, _SW)])

    @pl.when(c == 0)
    def _():
        pltpu.sync_copy(stripe_v, dis_hbm.at[pl.ds(sid * _SW, _SW)])
    plsc.subcore_barrier()

    pltpu.sync_copy(deg_sh, dis_v)

    def _s_super(ks, carry):
        base = ebase + ks * _SUP
        pltpu.sync_copy(src_hbm.at[pl.ds(base, _SUP)], src_v)
        pltpu.sync_copy(dst_hbm.at[pl.ds(base, _SUP)], dst_v)
        pltpu.sync_copy(w_hbm.at[pl.ds(base, _SUP)], w_v)

        def _val(g, carry2):
            d16 = dst_v[pl.ds(g * 16, 16)]
            s16 = src_v[pl.ds(g * 16, 16)]
            w16 = w_v[pl.ds(g * 16, 16)]
            disd = plsc.load_gather(dis_v, [d16])
            diss = plsc.load_gather(dis_v, [s16])
            val_v[pl.ds(g * 16, 16)] = w16 * disd
            wsc_v[pl.ds(g * 16, 16)] = w16 * diss
            return carry2
        lax.fori_loop(0, _G16, _val, None)

        @pl.when(c == 0)
        def _():
            pltpu.sync_copy(wsc_v, wsc_hbm.at[pl.ds(base, _SUP)])

        def _sub(j, carry2):
            def _cp(g, carry3):
                srcb[pl.ds(g * 16, 16)] = src_v[pl.ds(j * 128 + g * 16, 16)]
                return carry3
            lax.fori_loop(0, 8, _cp, None)
            pltpu.sync_copy(val_v.at[pl.ds(j * 128, 128)],
                            s_sh.at[srcb], add=True)
            return carry2
        lax.fori_loop(0, _SUBS, _sub, None)
        return carry
    lax.fori_loop(0, _NSUP, _s_super, None)
    plsc.subcore_barrier()

    @pl.when(c == 0)
    def _():
        pltpu.sync_copy(s_sh.at[pl.ds(sid * _SW, _SW)],
                        s_hbm.at[pl.ds(sid * _SW, _SW)])


_sc1_stage = functools.partial(
    pl.kernel,
    out_type=[
        jax.ShapeDtypeStruct((_NPAD,), jnp.float32),
        jax.ShapeDtypeStruct((_NPAD,), jnp.float32),
        jax.ShapeDtypeStruct((_EPAD,), jnp.float32),
    ],
    mesh=plsc.VectorSubcoreMesh(core_axis_name="c", subcore_axis_name="s",
                                num_cores=_NC, num_subcores=_NS),
    compiler_params=_SC_PARAMS,
    scratch_types=[
        pltpu.VMEM((_NPAD,), jnp.float32),
        pltpu.VMEM((_SUP,), jnp.int32),
        pltpu.VMEM((_SUP,), jnp.int32),
        pltpu.VMEM((_SUP,), jnp.float32),
        pltpu.VMEM((_SUP,), jnp.float32),
        pltpu.VMEM((_SUP,), jnp.float32),
        pltpu.VMEM((128,), jnp.int32),
        pltpu.VMEM((128,), jnp.int32),
        pltpu.VMEM((_SW,), jnp.float32),
        pltpu.VMEM_SHARED((_NPAD,), jnp.float32),
        pltpu.VMEM_SHARED((_NPAD,), jnp.float32),
    ],
)(_sc1_body)


def _lstm_body(x_ref, wih_ref, whhT_ref, b_ref, w1T_ref, out_ref):
    x = x_ref[...]
    wih = wih_ref[...]
    b = b_ref[...]
    whhT = whhT_ref[...]
    h = jnp.zeros((_BN, _H), jnp.float32)
    c = jnp.zeros((_BN, _H), jnp.float32)
    for t in range(_T):
        xt = x[:, t:t + 1]
        g = xt * wih + b
        g = g + jnp.dot(h, whhT, preferred_element_type=jnp.float32)
        i = 0.5 + 0.5 * jnp.tanh(0.5 * g[:, 0:_H])
        f = 0.5 + 0.5 * jnp.tanh(0.5 * g[:, _H:2 * _H])
        gg = jnp.tanh(g[:, 2 * _H:3 * _H])
        o = 0.5 + 0.5 * jnp.tanh(0.5 * g[:, 3 * _H:4 * _H])
        c = f * c + i * gg
        h = o * jnp.tanh(c)
    xw = jnp.dot(jax.nn.relu(h), w1T_ref[...],
                 preferred_element_type=jnp.float32)
    out_ref[0, :, :] = xw[:, 0:32]
    out_ref[1, :, :] = xw[:, 32:64]


def _lstm_stage(xpad, wih, whhT, b, w1T):
    return pl.pallas_call(
        _lstm_body,
        grid=(_GA,),
        in_specs=[
            pl.BlockSpec((_BN, _T), lambda i: (i, 0)),
            pl.BlockSpec((1, 4 * _H), lambda i: (0, 0)),
            pl.BlockSpec((_H, 4 * _H), lambda i: (0, 0)),
            pl.BlockSpec((1, 4 * _H), lambda i: (0, 0)),
            pl.BlockSpec((_H, _H), lambda i: (0, 0)),
        ],
        out_specs=pl.BlockSpec((2, _BN, 32), lambda i: (0, i, 0)),
        out_shape=jax.ShapeDtypeStruct((2, _NPAD, 32), jnp.float32),
    )(xpad, wih, whhT, b, w1T)


def _sc2_body(src_hbm, dst_hbm, w_hbm, xw1_hbm, dis_hbm, s_hbm, b1_hbm,
              vp_hbm,
              src_v, dst_v, w_v, rows_a, rows_b, dstb_a, dstb_b, pv,
              y1_sh, gsem, ssem):
    c = lax.axis_index("c")
    sid = lax.axis_index("s")
    ebase = sid * _EPT
    z16 = jnp.zeros((16,), jnp.float32)
    bufs = (rows_a, rows_b)
    dbufs = (dstb_a, dstb_b)
    xw1_page = xw1_hbm.at[c]

    def _zrows(e, carry):
        rows_a[e, pl.ds(0, 16)] = z16
        rows_a[e, pl.ds(16, 16)] = z16
        return carry
    lax.fori_loop(0, 128, _zrows, None)

    def _zy(k, carry):
        pltpu.sync_copy(rows_a, y1_sh.at[pl.ds(sid * _SW + k * 128, 128)])
        return carry
    lax.fori_loop(0, _SW // 128, _zy, None)
    pltpu.sync_copy(rows_a.at[pl.ds(0, 64)],
                    y1_sh.at[pl.ds(sid * _SW + (_SW // 128) * 128, 64)])
    plsc.subcore_barrier()

    def _main_super(ks, carry):
        base = ebase + ks * _SUP2
        pltpu.sync_copy(src_hbm.at[pl.ds(base, _SUP2)], src_v)
        pltpu.sync_copy(dst_hbm.at[pl.ds(base, _SUP2)], dst_v)
        pltpu.sync_copy(w_hbm.at[pl.ds(base, _SUP2)], w_v)

        sdesc = [None, None]
        gdesc = [None, None]
        gdesc[0] = pltpu.async_copy(
            xw1_page.at[src_v.at[pl.ds(0, 128)]], rows_a, gsem)
        for j in range(_SUBS2):
            p = j & 1
            q = 1 - p
            if j + 1 < _SUBS2:
                if sdesc[q] is not None:
                    sdesc[q].wait()
                    sdesc[q] = None
                gdesc[q] = pltpu.async_copy(
                    xw1_page.at[src_v.at[pl.ds((j + 1) * 128, 128)]],
                    bufs[q], gsem)
            gdesc[p].wait()

            def _scale(g, carry3, _j=j, _p=p):
                w16 = w_v[pl.ds(_j * 128 + g * 16, 16)]
                for k in range(16):
                    wk = w16[k]
                    r = bufs[_p]
                    e = g * 16 + k
                    r[e, pl.ds(0, 16)] = r[e, pl.ds(0, 16)] * wk
                    r[e, pl.ds(16, 16)] = r[e, pl.ds(16, 16)] * wk
                return carry3
            lax.fori_loop(0, 8, _scale, None)

            def _cp(g, carry3, _j=j, _p=p):
                dbufs[_p][pl.ds(g * 16, 16)] = \
                    dst_v[pl.ds(_j * 128 + g * 16, 16)]
                return carry3
            lax.fori_loop(0, 8, _cp, None)
            sdesc[p] = pltpu.async_copy(bufs[p], y1_sh.at[dbufs[p]],
                                        ssem, add=True)
        for d in sdesc:
            if d is not None:
                d.wait()
        return carry
    lax.fori_loop(0, _NSUP2, _main_super, None)
    plsc.subcore_barrier()

    nbase = sid * _SW
    pltpu.sync_copy(b1_hbm.at[c], pv)
    b1a = pv[pl.ds(0, 16)]
    b1b = pv[pl.ds(16, 16)]

    def _red(k, acc):
        rb = nbase + k * 112
        pltpu.sync_copy(y1_sh.at[pl.ds(rb, 112)], rows_a.at[pl.ds(0, 112)])
        pltpu.sync_copy(xw1_page.at[pl.ds(rb, 112)],
                        rows_b.at[pl.ds(0, 112)])
        pltpu.sync_copy(dis_hbm.at[pl.ds(rb, 112)], w_v.at[pl.ds(0, 112)])
        pltpu.sync_copy(s_hbm.at[pl.ds(rb, 112)], w_v.at[pl.ds(112, 112)])

        def _grp(g, acc2):
            a0, a1 = acc2
            d16 = w_v[pl.ds(g * 16, 16)]
            sv16 = w_v[pl.ds(112 + g * 16, 16)]
            ids = rb + g * 16 + lax.iota(jnp.int32, 16)
            a16 = jnp.where(ids < _N, d16 * (sv16 + d16), 0.0)
            for kk in range(16):
                e = g * 16 + kk
                dk = d16[kk]
                ak = a16[kk]
                y0 = rows_a[e, pl.ds(0, 16)]
                y1v = rows_a[e, pl.ds(16, 16)]
                x0 = rows_b[e, pl.ds(0, 16)]
                x1 = rows_b[e, pl.ds(16, 16)]
                z0 = jnp.maximum(dk * (y0 + dk * x0) + b1a, 0.0)
                z1 = jnp.maximum(dk * (y1v + dk * x1) + b1b, 0.0)
                a0 = a0 + ak * z0
                a1 = a1 + ak * z1
            return (a0, a1)
        return lax.fori_loop(0, 7, _grp, acc)
    acc0, acc1 = lax.fori_loop(0, _SW // 112, _red, (z16, z16))
    pv[pl.ds(0, 16)] = acc0
    pv[pl.ds(16, 16)] = acc1
    pltpu.sync_copy(pv, vp_hbm.at[c].at[sid])


_sc2_stage = functools.partial(
    pl.kernel,
    out_type=[
        jax.ShapeDtypeStruct((2, 16, 32), jnp.float32),
    ],
    mesh=plsc.VectorSubcoreMesh(core_axis_name="c", subcore_axis_name="s",
                                num_cores=_NC, num_subcores=_NS),
    compiler_params=_SC_PARAMS,
    scratch_types=[
        pltpu.VMEM((_SUP2,), jnp.int32),
        pltpu.VMEM((_SUP2,), jnp.int32),
        pltpu.VMEM((_SUP2,), jnp.float32),
        pltpu.VMEM((128, 32), jnp.float32),
        pltpu.VMEM((128, 32), jnp.float32),
        pltpu.VMEM((128,), jnp.int32),
        pltpu.VMEM((128,), jnp.int32),
        pltpu.VMEM((32,), jnp.float32),
        pltpu.VMEM_SHARED((_NPAD, 32), jnp.float32),
        pltpu.SemaphoreType.DMA,
        pltpu.SemaphoreType.DMA,
    ],
)(_sc2_body)


def _final_body(vp_ref, w2T_ref, b2_ref, out_ref):
    v0 = jnp.sum(vp_ref[0], axis=0, keepdims=True)
    v1 = jnp.sum(vp_ref[1], axis=0, keepdims=True)
    v = jnp.concatenate([v0, v1], axis=1)
    out_ref[...] = (jnp.dot(v, w2T_ref[...],
                            preferred_element_type=jnp.float32)
                    * np.float32(1.0 / _N) + b2_ref[...])


def _final_stage(vp, w2T, b2):
    return pl.pallas_call(
        _final_body,
        grid=(1,),
        in_specs=[
            pl.BlockSpec((2, 16, 32), lambda i: (0, 0, 0)),
            pl.BlockSpec((_H, _OUT), lambda i: (0, 0)),
            pl.BlockSpec((1, _OUT), lambda i: (0, 0)),
        ],
        out_specs=pl.BlockSpec((1, _OUT), lambda i: (0, 0)),
        out_shape=jax.ShapeDtypeStruct((1, _OUT), jnp.float32),
    )(vp, w2T, b2)


def kernel(node_features, edge_index, edge_attributes,
           W_ih, W_hh, b_ih, b_hh, W1, b1, W2, b2):
    f32 = jnp.float32
    xpad = jnp.zeros((_NPAD, _T), f32).at[:_N].set(node_features)
    wih = W_ih[:, 0][None, :]
    whhT = W_hh.T
    bsum = (b_ih + b_hh)[None, :]
    w1T = W1.T

    epad = _EPAD - _E
    zi = jnp.zeros((epad,), jnp.int32)
    srcp = jnp.concatenate([edge_index[0], zi])
    dstp = jnp.concatenate([edge_index[1], zi])
    wp = jnp.concatenate([edge_attributes, jnp.zeros((epad,), f32)])

    xw1p = _lstm_stage(xpad, wih, whhT, bsum, w1T)

    dis, s, wsc = _sc1_stage(srcp, dstp, wp)

    (vp,) = _sc2_stage(srcp, dstp, wsc, xw1p, dis, s, b1.reshape(2, 32))

    return _final_stage(vp, W2.T, b2[None, :])
